# Initial kernel scaffold; baseline (speedup 1.0000x reference)
import jax
import jax.numpy as jnp
from jax.experimental import pallas as pl


def kernel(x, edge_index, W1, b1, W2, b2):
    raise NotImplementedError("write your pallas kernel here")


# speedup vs baseline: 8.4708x; 8.4708x over previous
import jax
import jax.numpy as jnp
from jax import lax
from jax.experimental import pallas as pl
from jax.experimental.pallas import tpu as pltpu
from jax.experimental.pallas import tpu_sc as plsc

N = 10000
D_IN = 128
D_HID = 256
D_OUT = 128
E = 320000
R = 10240
CHUNK = 128
NC, NS = 2, 16
EPAD = 323584
HALF = R // 2
F32 = jnp.float32

_MESH = plsc.VectorSubcoreMesh(core_axis_name="c", subcore_axis_name="s")


def _deg_body(dst_hbm, deg0_hbm, deg1_hbm, idx_v, acc_v, red_v, tmp_v,
              stage_sh):
    c = lax.axis_index("c")
    s = lax.axis_index("s")
    nchunks = EPAD // (NC * NS * CHUNK)
    lanes = lax.iota(jnp.int32, 16)
    ones = jnp.ones((16,), F32)

    for p in range(2):
        lo = p * HALF

        def zero_acc(i, _):
            acc_v[pl.ds(i * 16, 16)] = jnp.zeros((16,), F32)
            return 0
        lax.fori_loop(0, (16 * HALF) // 16, zero_acc, 0)

        def chunk_body(i, _):
            base = ((c * NS + s) * nchunks + i) * CHUNK
            pltpu.sync_copy(dst_hbm.at[pl.ds(base, CHUNK)], idx_v)
            for k in range(CHUNK // 16):
                v = idx_v[pl.ds(k * 16, 16)]
                rel = v - lo
                m = (rel >= 0) & (rel < HALF)
                rel_c = jnp.clip(rel, 0, HALF - 1)
                plsc.addupdate_scatter(acc_v, [rel_c + lanes * HALF], ones,
                                       mask=m)
            return 0
        lax.fori_loop(0, nchunks, chunk_body, 0)

        def red_body(j, _):
            t = acc_v[pl.ds(j * 16, 16)]
            for l in range(1, 16):
                t = t + acc_v[pl.ds(l * HALF + j * 16, 16)]
            red_v[pl.ds(lo + j * 16, 16)] = t
            return 0
        lax.fori_loop(0, HALF // 16, red_body, 0)

    pltpu.sync_copy(red_v, stage_sh.at[s])
    plsc.subcore_barrier()
    rows = R // NS

    def zero_red(i, _):
        red_v[pl.ds(i * 16, 16)] = jnp.zeros((16,), F32)
        return 0
    lax.fori_loop(0, rows // 16, zero_red, 0)
    for l in range(NS):
        pltpu.sync_copy(stage_sh.at[l, pl.ds(s * rows, rows)], tmp_v)

        def add_body(j, _):
            red_v[pl.ds(j * 16, 16)] = (red_v[pl.ds(j * 16, 16)]
                                        + tmp_v[pl.ds(j * 16, 16)])
            return 0
        lax.fori_loop(0, rows // 16, add_body, 0)

    @pl.when(c == 0)
    def _():
        pltpu.sync_copy(red_v.at[pl.ds(0, rows)],
                        deg0_hbm.at[pl.ds(s * rows, rows)])

    @pl.when(c == 1)
    def _():
        pltpu.sync_copy(red_v.at[pl.ds(0, rows)],
                        deg1_hbm.at[pl.ds(s * rows, rows)])


_deg_call = pl.kernel(
    _deg_body,
    out_type=[jax.ShapeDtypeStruct((R,), F32),
              jax.ShapeDtypeStruct((R,), F32)],
    mesh=_MESH,
    compiler_params=pltpu.CompilerParams(needs_layout_passes=False),
    scratch_types=[
        pltpu.VMEM((CHUNK,), jnp.int32),
        pltpu.VMEM((16 * HALF,), F32),
        pltpu.VMEM((R,), F32),
        pltpu.VMEM((R // NS,), F32),
        pltpu.VMEM_SHARED((NS, R), F32),
    ],
)


def _zero_spmem_slice(zv, acc_sh, s):
    rows = R // NS
    for k in range(rows // 64):
        pltpu.sync_copy(zv, acc_sh.at[pl.ds(s * rows + k * 64, 64)])


def _fill_zeros2d(ref, rows, cols):
    def body(i, _):
        for j in range(cols // 16):
            ref[i, pl.ds(j * 16, 16)] = jnp.zeros((16,), F32)
        return 0
    lax.fori_loop(0, rows, body, 0)


def _agg_chunks(src_hbm, dst_hbm, ht_hbm, acc_sh, idxs_v, idxd_v, gbuf_v,
                sem, nchunks, wid):
    def chunk_body(i, _):
        base = (wid * nchunks + i) * CHUNK
        pltpu.sync_copy(src_hbm.at[pl.ds(base, CHUNK)], idxs_v)
        pltpu.async_copy(ht_hbm.at[idxs_v], gbuf_v, sem).wait()
        pltpu.sync_copy(dst_hbm.at[pl.ds(base, CHUNK)], idxd_v)
        pltpu.sync_copy(gbuf_v, acc_sh.at[idxd_v], add=True)
        return 0
    lax.fori_loop(0, nchunks, chunk_body, 0)


def _agg1_body(htl_hbm, htr_hbm, src_hbm, dst_hbm, aggl_hbm, aggr_hbm,
               idxs_v, idxd_v, gbuf_v, zv, acc_sh, sem):
    c = lax.axis_index("c")
    s = lax.axis_index("s")
    _fill_zeros2d(zv, 64, CHUNK)
    _zero_spmem_slice(zv, acc_sh, s)
    plsc.subcore_barrier()
    nchunks = EPAD // (NS * CHUNK)

    @pl.when(c == 0)
    def _():
        _agg_chunks(src_hbm, dst_hbm, htl_hbm, acc_sh, idxs_v, idxd_v,
                    gbuf_v, sem, nchunks, s)

    @pl.when(c == 1)
    def _():
        _agg_chunks(src_hbm, dst_hbm, htr_hbm, acc_sh, idxs_v, idxd_v,
                    gbuf_v, sem, nchunks, s)

    plsc.subcore_barrier()
    rows = R // NS

    @pl.when(c == 0)
    def _():
        pltpu.sync_copy(acc_sh.at[pl.ds(s * rows, rows)],
                        aggl_hbm.at[pl.ds(s * rows, rows)])

    @pl.when(c == 1)
    def _():
        pltpu.sync_copy(acc_sh.at[pl.ds(s * rows, rows)],
                        aggr_hbm.at[pl.ds(s * rows, rows)])


_agg1_call = pl.kernel(
    _agg1_body,
    out_type=[jax.ShapeDtypeStruct((R, D_HID // 2), F32),
              jax.ShapeDtypeStruct((R, D_HID // 2), F32)],
    mesh=_MESH,
    scratch_types=[
        pltpu.VMEM((CHUNK,), jnp.int32),
        pltpu.VMEM((CHUNK,), jnp.int32),
        pltpu.VMEM((CHUNK, D_HID // 2), F32),
        pltpu.VMEM((64, CHUNK), F32),
        pltpu.VMEM_SHARED((R, D_HID // 2), F32),
        pltpu.SemaphoreType.DMA,
    ],
)


def _agg2_body(ht_hbm, src_hbm, dst_hbm, agg0_hbm, agg1_hbm,
               idxs_v, idxd_v, gbuf_v, zv, acc_sh, sem):
    c = lax.axis_index("c")
    s = lax.axis_index("s")
    _fill_zeros2d(zv, 64, D_OUT)
    _zero_spmem_slice(zv, acc_sh, s)
    plsc.subcore_barrier()
    nchunks = EPAD // (NC * NS * CHUNK)
    _agg_chunks(src_hbm, dst_hbm, ht_hbm, acc_sh, idxs_v, idxd_v,
                gbuf_v, sem, nchunks, c * NS + s)
    plsc.subcore_barrier()
    rows = R // NS

    @pl.when(c == 0)
    def _():
        pltpu.sync_copy(acc_sh.at[pl.ds(s * rows, rows)],
                        agg0_hbm.at[pl.ds(s * rows, rows)])

    @pl.when(c == 1)
    def _():
        pltpu.sync_copy(acc_sh.at[pl.ds(s * rows, rows)],
                        agg1_hbm.at[pl.ds(s * rows, rows)])


_agg2_call = pl.kernel(
    _agg2_body,
    out_type=[jax.ShapeDtypeStruct((R, D_OUT), F32),
              jax.ShapeDtypeStruct((R, D_OUT), F32)],
    mesh=_MESH,
    scratch_types=[
        pltpu.VMEM((CHUNK,), jnp.int32),
        pltpu.VMEM((CHUNK,), jnp.int32),
        pltpu.VMEM((CHUNK, D_OUT), F32),
        pltpu.VMEM((64, D_OUT), F32),
        pltpu.VMEM_SHARED((R, D_OUT), F32),
        pltpu.SemaphoreType.DMA,
    ],
)


_BR = 256
_GRID = R // _BR


def _k1_body(x_ref, w1_ref, d0_ref, d1_ref, htl_ref, htr_ref, dis_ref):
    deg = d0_ref[...] + d1_ref[...] + 1.0
    dis = lax.rsqrt(deg)
    dis_ref[...] = dis
    h = jnp.dot(x_ref[...], w1_ref[...], preferred_element_type=F32)
    ht = h * dis
    htl_ref[...] = ht[:, :D_HID // 2]
    htr_ref[...] = ht[:, D_HID // 2:]


_k1_call = pl.pallas_call(
    _k1_body,
    grid=(_GRID,),
    in_specs=[
        pl.BlockSpec((_BR, D_IN), lambda i: (i, 0)),
        pl.BlockSpec((D_IN, D_HID), lambda i: (0, 0)),
        pl.BlockSpec((_BR, 1), lambda i: (i, 0)),
        pl.BlockSpec((_BR, 1), lambda i: (i, 0)),
    ],
    out_specs=[
        pl.BlockSpec((_BR, D_HID // 2), lambda i: (i, 0)),
        pl.BlockSpec((_BR, D_HID // 2), lambda i: (i, 0)),
        pl.BlockSpec((_BR, 1), lambda i: (i, 0)),
    ],
    out_shape=[
        jax.ShapeDtypeStruct((R, D_HID // 2), F32),
        jax.ShapeDtypeStruct((R, D_HID // 2), F32),
        jax.ShapeDtypeStruct((R, 1), F32),
    ],
)


def _k2_body(al_ref, ar_ref, hl_ref, hr_ref, dis_ref, b1_ref, w2_ref,
             hrelu_ref, ht2_ref):
    dis = dis_ref[...]
    s1 = (al_ref[...] + hl_ref[...]) * dis
    s2 = (ar_ref[...] + hr_ref[...]) * dis
    h1 = jnp.concatenate([s1, s2], axis=1) + b1_ref[...]
    hr = jnp.maximum(h1, 0.0)
    hrelu_ref[...] = hr
    h2 = jnp.dot(hr, w2_ref[...], preferred_element_type=F32)
    ht2_ref[...] = h2 * dis


_k2_call = pl.pallas_call(
    _k2_body,
    grid=(_GRID,),
    in_specs=[
        pl.BlockSpec((_BR, D_HID // 2), lambda i: (i, 0)),
        pl.BlockSpec((_BR, D_HID // 2), lambda i: (i, 0)),
        pl.BlockSpec((_BR, D_HID // 2), lambda i: (i, 0)),
        pl.BlockSpec((_BR, D_HID // 2), lambda i: (i, 0)),
        pl.BlockSpec((_BR, 1), lambda i: (i, 0)),
        pl.BlockSpec((1, D_HID), lambda i: (0, 0)),
        pl.BlockSpec((D_HID, D_OUT), lambda i: (0, 0)),
    ],
    out_specs=[
        pl.BlockSpec((_BR, D_HID), lambda i: (i, 0)),
        pl.BlockSpec((_BR, D_OUT), lambda i: (i, 0)),
    ],
    out_shape=[
        jax.ShapeDtypeStruct((R, D_HID), F32),
        jax.ShapeDtypeStruct((R, D_OUT), F32),
    ],
)


def _k3_body(a0_ref, a1_ref, ht2_ref, dis_ref, b2_ref, out_ref):
    s = a0_ref[...] + a1_ref[...] + ht2_ref[...]
    out_ref[...] = s * dis_ref[...] + b2_ref[...]


_k3_call = pl.pallas_call(
    _k3_body,
    grid=(_GRID,),
    in_specs=[
        pl.BlockSpec((_BR, D_OUT), lambda i: (i, 0)),
        pl.BlockSpec((_BR, D_OUT), lambda i: (i, 0)),
        pl.BlockSpec((_BR, D_OUT), lambda i: (i, 0)),
        pl.BlockSpec((_BR, 1), lambda i: (i, 0)),
        pl.BlockSpec((1, D_OUT), lambda i: (0, 0)),
    ],
    out_specs=pl.BlockSpec((_BR, D_OUT), lambda i: (i, 0)),
    out_shape=jax.ShapeDtypeStruct((R, D_OUT), F32),
)


def kernel(x, edge_index, W1, b1, W2, b2):
    pad_e = EPAD - E
    src = jnp.concatenate([edge_index[0],
                           jnp.full((pad_e,), N, edge_index.dtype)])
    dst = jnp.concatenate([edge_index[1],
                           jnp.full((pad_e,), N, edge_index.dtype)])
    x_pad = jnp.zeros((R, D_IN), F32).at[:N].set(x)

    deg0, deg1 = _deg_call(dst)
    htl, htr, dis = _k1_call(x_pad, W1, deg0.reshape(R, 1),
                             deg1.reshape(R, 1))
    a1l, a1r = _agg1_call(htl, htr, src, dst)
    hrelu, ht2 = _k2_call(a1l, a1r, htl, htr, dis, b1.reshape(1, -1), W2)
    a20, a21 = _agg2_call(ht2, src, dst)
    out2 = _k3_call(a20, a21, ht2, dis, b2.reshape(1, -1))
    return out2[:N], hrelu[:N]


# speedup vs baseline: 8.8680x; 1.0469x over previous
import jax
import jax.numpy as jnp
from jax import lax
from jax.experimental import pallas as pl
from jax.experimental.pallas import tpu as pltpu
from jax.experimental.pallas import tpu_sc as plsc

N = 10000
D_IN = 128
D_HID = 256
D_OUT = 128
E = 320000
R = 10240
CHUNK = 128
NC, NS = 2, 16
EROWS = 2560
EPAD = EROWS * CHUNK
NBUF = 4
HALF = R // 2
F32 = jnp.float32

_MESH = plsc.VectorSubcoreMesh(core_axis_name="c", subcore_axis_name="s")


def _deg_body(dst_hbm, deg0_hbm, deg1_hbm, idx_v, acc_v, red_v, tmp_v,
              stage_sh):
    c = lax.axis_index("c")
    s = lax.axis_index("s")
    nrows = EROWS // (NC * NS)
    lanes = lax.iota(jnp.int32, 16)
    lane_off = lanes * HALF
    ones = jnp.ones((16,), F32)
    pltpu.sync_copy(dst_hbm.at[pl.ds((c * NS + s) * nrows, nrows)], idx_v)

    for p in range(2):
        lo = p * HALF

        def zero_acc(i, _):
            acc_v[pl.ds(i * 16, 16)] = jnp.zeros((16,), F32)
            return 0
        lax.fori_loop(0, (16 * HALF) // 16, zero_acc, 0)

        def row_body(i, _):
            for k in range(CHUNK // 16):
                v = idx_v[i, pl.ds(k * 16, 16)]
                rel = v - lo
                m = (rel >= 0) & (rel < HALF)
                rel_c = jnp.clip(rel, 0, HALF - 1)
                plsc.addupdate_scatter(acc_v, [rel_c + lane_off], ones,
                                       mask=m)
            return 0
        lax.fori_loop(0, nrows, row_body, 0)

        def red_body(j, _):
            t = acc_v[pl.ds(j * 16, 16)]
            for l in range(1, 16):
                t = t + acc_v[pl.ds(l * HALF + j * 16, 16)]
            red_v[pl.ds(lo + j * 16, 16)] = t
            return 0
        lax.fori_loop(0, HALF // 16, red_body, 0)

    pltpu.sync_copy(red_v, stage_sh.at[s])
    plsc.subcore_barrier()
    rows = R // NS

    def zero_red(i, _):
        red_v[pl.ds(i * 16, 16)] = jnp.zeros((16,), F32)
        return 0
    lax.fori_loop(0, rows // 16, zero_red, 0)
    for l in range(NS):
        pltpu.sync_copy(stage_sh.at[l, pl.ds(s * rows, rows)], tmp_v)

        def add_body(j, _):
            red_v[pl.ds(j * 16, 16)] = (red_v[pl.ds(j * 16, 16)]
                                        + tmp_v[pl.ds(j * 16, 16)])
            return 0
        lax.fori_loop(0, rows // 16, add_body, 0)

    @pl.when(c == 0)
    def _():
        pltpu.sync_copy(red_v.at[pl.ds(0, rows)],
                        deg0_hbm.at[pl.ds(s * rows, rows)])

    @pl.when(c == 1)
    def _():
        pltpu.sync_copy(red_v.at[pl.ds(0, rows)],
                        deg1_hbm.at[pl.ds(s * rows, rows)])


_deg_call = pl.kernel(
    _deg_body,
    name='degk',
    out_type=[jax.ShapeDtypeStruct((R,), F32),
              jax.ShapeDtypeStruct((R,), F32)],
    mesh=_MESH,
    compiler_params=pltpu.CompilerParams(needs_layout_passes=False),
    scratch_types=[
        pltpu.VMEM((EROWS // (NC * NS), CHUNK), jnp.int32),
        pltpu.VMEM((16 * HALF,), F32),
        pltpu.VMEM((R,), F32),
        pltpu.VMEM((R // NS,), F32),
        pltpu.VMEM_SHARED((NS, R), F32),
    ],
)


def _fill_zeros2d(ref, rows, cols):
    def body(i, _):
        for j in range(cols // 16):
            ref[i, pl.ds(j * 16, 16)] = jnp.zeros((16,), F32)
        return 0
    lax.fori_loop(0, rows, body, 0)


def _idx_wait(src_hbm, sb, db, semi):
    pltpu.make_async_copy(src_hbm.at[pl.ds(0, CHUNK)], sb, semi).wait()
    pltpu.make_async_copy(src_hbm.at[pl.ds(0, CHUNK)], db, semi).wait()


def _agg_body_common(ht_hbm, src_hbm, dst_hbm, acc_sh, gbufs, sbufs, dbufs,
                     semg, semi, s, row0, nrows):
    _fill_zeros2d(gbufs[0], CHUNK, gbufs[0].shape[1])
    rows = R // NS
    zdescs = [pltpu.make_async_copy(
        gbufs[0], acc_sh.at[pl.ds(s * rows + k * CHUNK, CHUNK)], semg[0])
        for k in range(rows // CHUNK)]
    for d in zdescs:
        d.start()
    for d in zdescs:
        d.wait()
    plsc.subcore_barrier()

    for tslot in range(4):
        pltpu.async_copy(src_hbm.at[pl.ds((row0 + tslot) * CHUNK, CHUNK)],
                         sbufs[tslot], semi[tslot])
        pltpu.async_copy(dst_hbm.at[pl.ds((row0 + tslot) * CHUNK, CHUNK)],
                         dbufs[tslot], semi[tslot])
    for bg in range(2):
        _idx_wait(src_hbm, sbufs[bg], dbufs[bg], semi[bg])
        pltpu.async_copy(ht_hbm.at[sbufs[bg]], gbufs[bg], semg[bg])

    nsteps = nrows // 4

    def step(g, _):
        for b4 in range(4):
            i = g * 4 + b4
            gi = b4 % 2
            s2 = (b4 + 2) % 4
            pltpu.make_async_copy(ht_hbm.at[sbufs[b4]], gbufs[gi],
                                  semg[gi]).wait()
            pltpu.sync_copy(gbufs[gi], acc_sh.at[dbufs[b4]], add=True)
            @pl.when(g < nsteps - 1)
            def _():
                pltpu.async_copy(
                    src_hbm.at[pl.ds((row0 + i + 4) * CHUNK, CHUNK)],
                    sbufs[b4], semi[b4])
                pltpu.async_copy(
                    dst_hbm.at[pl.ds((row0 + i + 4) * CHUNK, CHUNK)],
                    dbufs[b4], semi[b4])
            if b4 < 2:
                _idx_wait(src_hbm, sbufs[s2], dbufs[s2], semi[s2])
                pltpu.async_copy(ht_hbm.at[sbufs[s2]], gbufs[gi],
                                 semg[gi])
            else:
                @pl.when(g < nsteps - 1)
                def _():
                    _idx_wait(src_hbm, sbufs[s2], dbufs[s2], semi[s2])
                    pltpu.async_copy(ht_hbm.at[sbufs[s2]], gbufs[gi],
                                     semg[gi])
        return 0
    lax.fori_loop(0, nsteps, step, 0)


def _agg_epilogue(acc_sh, out_hbm, s):
    rows = R // NS
    pltpu.sync_copy(acc_sh.at[pl.ds(s * rows, rows)],
                    out_hbm.at[pl.ds(s * rows, rows)])


def _agg1_body(htl_hbm, htr_hbm, src_hbm, dst_hbm, aggl_hbm, aggr_hbm,
               gb0, gb1, sb0, sb1, sb2, sb3, db0, db1, db2, db3, acc_sh,
               smg0, smg1, smi0, smi1, smi2, smi3):
    c = lax.axis_index("c")
    s = lax.axis_index("s")
    nrows = EROWS // NS
    gbufs = (gb0, gb1)
    sbufs = (sb0, sb1, sb2, sb3)
    dbufs = (db0, db1, db2, db3)
    semg = (smg0, smg1)
    semi = (smi0, smi1, smi2, smi3)

    @pl.when(c == 0)
    def _():
        _agg_body_common(htl_hbm, src_hbm, dst_hbm, acc_sh, gbufs, sbufs,
                         dbufs, semg, semi, s, s * nrows, nrows)

    @pl.when(c == 1)
    def _():
        _agg_body_common(htr_hbm, src_hbm, dst_hbm, acc_sh, gbufs, sbufs,
                         dbufs, semg, semi, s, s * nrows, nrows)

    plsc.subcore_barrier()

    @pl.when(c == 0)
    def _():
        _agg_epilogue(acc_sh, aggl_hbm, s)

    @pl.when(c == 1)
    def _():
        _agg_epilogue(acc_sh, aggr_hbm, s)


def _agg_scratch(dsc):
    return [
        pltpu.VMEM((CHUNK, dsc), F32),
        pltpu.VMEM((CHUNK, dsc), F32),
        pltpu.VMEM((CHUNK,), jnp.int32),
        pltpu.VMEM((CHUNK,), jnp.int32),
        pltpu.VMEM((CHUNK,), jnp.int32),
        pltpu.VMEM((CHUNK,), jnp.int32),
        pltpu.VMEM((CHUNK,), jnp.int32),
        pltpu.VMEM((CHUNK,), jnp.int32),
        pltpu.VMEM((CHUNK,), jnp.int32),
        pltpu.VMEM((CHUNK,), jnp.int32),
        pltpu.VMEM_SHARED((R, dsc), F32),
        pltpu.SemaphoreType.DMA,
        pltpu.SemaphoreType.DMA,
        pltpu.SemaphoreType.DMA,
        pltpu.SemaphoreType.DMA,
        pltpu.SemaphoreType.DMA,
        pltpu.SemaphoreType.DMA,
    ]


_agg1_call = pl.kernel(
    _agg1_body,
    name='agg1k',
    out_type=[jax.ShapeDtypeStruct((R, D_HID // 2), F32),
              jax.ShapeDtypeStruct((R, D_HID // 2), F32)],
    mesh=_MESH,
    scratch_types=_agg_scratch(D_HID // 2),
)


def _agg2_body(ht_hbm, src_hbm, dst_hbm, agg0_hbm, agg1_hbm,
               gb0, gb1, sb0, sb1, sb2, sb3, db0, db1, db2, db3, acc_sh,
               smg0, smg1, smi0, smi1, smi2, smi3):
    c = lax.axis_index("c")
    s = lax.axis_index("s")
    nrows = EROWS // (NC * NS)
    _agg_body_common(ht_hbm, src_hbm, dst_hbm, acc_sh,
                     (gb0, gb1), (sb0, sb1, sb2, sb3),
                     (db0, db1, db2, db3), (smg0, smg1),
                     (smi0, smi1, smi2, smi3), s,
                     (c * NS + s) * nrows, nrows)
    plsc.subcore_barrier()

    @pl.when(c == 0)
    def _():
        _agg_epilogue(acc_sh, agg0_hbm, s)

    @pl.when(c == 1)
    def _():
        _agg_epilogue(acc_sh, agg1_hbm, s)


_agg2_call = pl.kernel(
    _agg2_body,
    name='agg2k',
    out_type=[jax.ShapeDtypeStruct((R, D_OUT), F32),
              jax.ShapeDtypeStruct((R, D_OUT), F32)],
    mesh=_MESH,
    scratch_types=_agg_scratch(D_OUT),
)


_BR = 256
_GRID = R // _BR


def _k1_body(x_ref, w1_ref, d0_ref, d1_ref, htl_ref, htr_ref, dis_ref):
    deg = d0_ref[...] + d1_ref[...] + 1.0
    dis = lax.rsqrt(deg)
    dis_ref[...] = dis
    h = jnp.dot(x_ref[...], w1_ref[...], preferred_element_type=F32)
    ht = h * dis
    htl_ref[...] = ht[:, :D_HID // 2]
    htr_ref[...] = ht[:, D_HID // 2:]


_k1_call = pl.pallas_call(
    _k1_body,
    grid=(_GRID,),
    in_specs=[
        pl.BlockSpec((_BR, D_IN), lambda i: (i, 0)),
        pl.BlockSpec((D_IN, D_HID), lambda i: (0, 0)),
        pl.BlockSpec((_BR, 1), lambda i: (i, 0)),
        pl.BlockSpec((_BR, 1), lambda i: (i, 0)),
    ],
    out_specs=[
        pl.BlockSpec((_BR, D_HID // 2), lambda i: (i, 0)),
        pl.BlockSpec((_BR, D_HID // 2), lambda i: (i, 0)),
        pl.BlockSpec((_BR, 1), lambda i: (i, 0)),
    ],
    out_shape=[
        jax.ShapeDtypeStruct((R, D_HID // 2), F32),
        jax.ShapeDtypeStruct((R, D_HID // 2), F32),
        jax.ShapeDtypeStruct((R, 1), F32),
    ],
)


def _k2_body(al_ref, ar_ref, hl_ref, hr_ref, dis_ref, b1_ref, w2_ref,
             hrelu_ref, ht2_ref):
    dis = dis_ref[...]
    s1 = (al_ref[...] + hl_ref[...]) * dis
    s2 = (ar_ref[...] + hr_ref[...]) * dis
    h1 = jnp.concatenate([s1, s2], axis=1) + b1_ref[...]
    hr = jnp.maximum(h1, 0.0)
    hrelu_ref[...] = hr
    h2 = jnp.dot(hr, w2_ref[...], preferred_element_type=F32)
    ht2_ref[...] = h2 * dis


_k2_call = pl.pallas_call(
    _k2_body,
    grid=(_GRID,),
    in_specs=[
        pl.BlockSpec((_BR, D_HID // 2), lambda i: (i, 0)),
        pl.BlockSpec((_BR, D_HID // 2), lambda i: (i, 0)),
        pl.BlockSpec((_BR, D_HID // 2), lambda i: (i, 0)),
        pl.BlockSpec((_BR, D_HID // 2), lambda i: (i, 0)),
        pl.BlockSpec((_BR, 1), lambda i: (i, 0)),
        pl.BlockSpec((1, D_HID), lambda i: (0, 0)),
        pl.BlockSpec((D_HID, D_OUT), lambda i: (0, 0)),
    ],
    out_specs=[
        pl.BlockSpec((_BR, D_HID), lambda i: (i, 0)),
        pl.BlockSpec((_BR, D_OUT), lambda i: (i, 0)),
    ],
    out_shape=[
        jax.ShapeDtypeStruct((R, D_HID), F32),
        jax.ShapeDtypeStruct((R, D_OUT), F32),
    ],
)


def _k3_body(a0_ref, a1_ref, ht2_ref, dis_ref, b2_ref, out_ref):
    s = a0_ref[...] + a1_ref[...] + ht2_ref[...]
    out_ref[...] = s * dis_ref[...] + b2_ref[...]


_k3_call = pl.pallas_call(
    _k3_body,
    grid=(_GRID,),
    in_specs=[
        pl.BlockSpec((_BR, D_OUT), lambda i: (i, 0)),
        pl.BlockSpec((_BR, D_OUT), lambda i: (i, 0)),
        pl.BlockSpec((_BR, D_OUT), lambda i: (i, 0)),
        pl.BlockSpec((_BR, 1), lambda i: (i, 0)),
        pl.BlockSpec((1, D_OUT), lambda i: (0, 0)),
    ],
    out_specs=pl.BlockSpec((_BR, D_OUT), lambda i: (i, 0)),
    out_shape=jax.ShapeDtypeStruct((R, D_OUT), F32),
)


def kernel(x, edge_index, W1, b1, W2, b2):
    pad_e = EPAD - E
    src = jnp.concatenate([edge_index[0],
                           jnp.full((pad_e,), N, edge_index.dtype)])
    dst = jnp.concatenate([edge_index[1],
                           jnp.full((pad_e,), N, edge_index.dtype)])
    src2 = src.reshape(EROWS, CHUNK)
    dst2 = dst.reshape(EROWS, CHUNK)
    x_pad = jnp.zeros((R, D_IN), F32).at[:N].set(x)

    deg0, deg1 = _deg_call(dst2)
    htl, htr, dis = _k1_call(x_pad, W1, deg0.reshape(R, 1),
                             deg1.reshape(R, 1))
    a1l, a1r = _agg1_call(htl, htr, src, dst)
    hrelu, ht2 = _k2_call(a1l, a1r, htl, htr, dis, b1.reshape(1, -1), W2)
    a20, a21 = _agg2_call(ht2, src, dst)
    out2 = _k3_call(a20, a21, ht2, dis, b2.reshape(1, -1))
    return out2[:N], hrelu[:N]


# speedup vs baseline: 21.4531x; 2.4192x over previous
import jax
import jax.numpy as jnp
from jax import lax
from jax.experimental import pallas as pl
from jax.experimental.pallas import tpu as pltpu
from jax.experimental.pallas import tpu_sc as plsc

N = 10000
D_IN = 128
D_HID = 256
D_OUT = 128
E = 320000
R = 10240
CHUNK = 128
NC, NS = 2, 16
EROWS = 2560
EPAD = EROWS * CHUNK
NBUF = 4
HALF = R // 2
F32 = jnp.float32

_MESH = plsc.VectorSubcoreMesh(core_axis_name="c", subcore_axis_name="s")


def _deg_body(dst_hbm, deg0_hbm, deg1_hbm, idx_v, acc_v, red_v, tmp_v,
              stage_sh):
    c = lax.axis_index("c")
    s = lax.axis_index("s")
    nrows = EROWS // (NC * NS)
    lanes = lax.iota(jnp.int32, 16)
    lane_off = lanes * HALF
    ones = jnp.ones((16,), F32)
    pltpu.sync_copy(dst_hbm.at[pl.ds((c * NS + s) * nrows, nrows)], idx_v)

    for p in range(2):
        lo = p * HALF

        def zero_acc(i, _):
            acc_v[pl.ds(i * 16, 16)] = jnp.zeros((16,), F32)
            return 0
        lax.fori_loop(0, (16 * HALF) // 16, zero_acc, 0)

        def row_body(i, _):
            for k in range(CHUNK // 16):
                v = idx_v[i, pl.ds(k * 16, 16)]
                rel = v - lo
                m = (rel >= 0) & (rel < HALF)
                rel_c = jnp.clip(rel, 0, HALF - 1)
                plsc.addupdate_scatter(acc_v, [rel_c + lane_off], ones,
                                       mask=m)
            return 0
        lax.fori_loop(0, nrows, row_body, 0)

        def red_body(j, _):
            t = acc_v[pl.ds(j * 16, 16)]
            for l in range(1, 16):
                t = t + acc_v[pl.ds(l * HALF + j * 16, 16)]
            red_v[pl.ds(lo + j * 16, 16)] = t
            return 0
        lax.fori_loop(0, HALF // 16, red_body, 0)

    pltpu.sync_copy(red_v, stage_sh.at[s])
    plsc.subcore_barrier()
    rows = R // NS

    def zero_red(i, _):
        red_v[pl.ds(i * 16, 16)] = jnp.zeros((16,), F32)
        return 0
    lax.fori_loop(0, rows // 16, zero_red, 0)
    for l in range(NS):
        pltpu.sync_copy(stage_sh.at[l, pl.ds(s * rows, rows)], tmp_v)

        def add_body(j, _):
            red_v[pl.ds(j * 16, 16)] = (red_v[pl.ds(j * 16, 16)]
                                        + tmp_v[pl.ds(j * 16, 16)])
            return 0
        lax.fori_loop(0, rows // 16, add_body, 0)

    @pl.when(c == 0)
    def _():
        pltpu.sync_copy(red_v.at[pl.ds(0, rows)],
                        deg0_hbm.at[pl.ds(s * rows, rows)])

    @pl.when(c == 1)
    def _():
        pltpu.sync_copy(red_v.at[pl.ds(0, rows)],
                        deg1_hbm.at[pl.ds(s * rows, rows)])


_deg_call = pl.kernel(
    _deg_body,
    name='degk',
    out_type=[jax.ShapeDtypeStruct((R,), F32),
              jax.ShapeDtypeStruct((R,), F32)],
    mesh=_MESH,
    compiler_params=pltpu.CompilerParams(needs_layout_passes=False),
    scratch_types=[
        pltpu.VMEM((EROWS // (NC * NS), CHUNK), jnp.int32),
        pltpu.VMEM((16 * HALF,), F32),
        pltpu.VMEM((R,), F32),
        pltpu.VMEM((R // NS,), F32),
        pltpu.VMEM_SHARED((NS, R), F32),
    ],
)


def _fill_zeros2d(ref, rows, cols):
    def body(i, _):
        for j in range(cols // 16):
            ref[i, pl.ds(j * 16, 16)] = jnp.zeros((16,), F32)
        return 0
    lax.fori_loop(0, rows, body, 0)


def _idx_wait(src_hbm, sb, db, semi):
    pltpu.make_async_copy(src_hbm.at[pl.ds(0, CHUNK)], sb, semi).wait()
    pltpu.make_async_copy(src_hbm.at[pl.ds(0, CHUNK)], db, semi).wait()


def _agg_body_common(ht_hbm, src_hbm, dst_hbm, acc_sh, gbufs, sbufs, dbufs,
                     semg, semi, s, row0, nrows):
    _fill_zeros2d(gbufs[0], CHUNK, gbufs[0].shape[1])
    rows = R // NS
    zdescs = [pltpu.make_async_copy(
        gbufs[0], acc_sh.at[pl.ds(s * rows + k * CHUNK, CHUNK)], semg[0])
        for k in range(rows // CHUNK)]
    for d in zdescs:
        d.start()
    for d in zdescs:
        d.wait()
    plsc.subcore_barrier()

    for tslot in range(4):
        pltpu.async_copy(src_hbm.at[pl.ds((row0 + tslot) * CHUNK, CHUNK)],
                         sbufs[tslot], semi[tslot])
        pltpu.async_copy(dst_hbm.at[pl.ds((row0 + tslot) * CHUNK, CHUNK)],
                         dbufs[tslot], semi[tslot])
    for bg in range(2):
        _idx_wait(src_hbm, sbufs[bg], dbufs[bg], semi[bg])
        pltpu.async_copy(ht_hbm.at[sbufs[bg]], gbufs[bg], semg[bg])

    nsteps = nrows // 4

    def step(g, _):
        for b4 in range(4):
            i = g * 4 + b4
            gi = b4 % 2
            s2 = (b4 + 2) % 4
            pltpu.make_async_copy(ht_hbm.at[sbufs[b4]], gbufs[gi],
                                  semg[gi]).wait()
            pltpu.sync_copy(gbufs[gi], acc_sh.at[dbufs[b4]], add=True)
            @pl.when(g < nsteps - 1)
            def _():
                pltpu.async_copy(
                    src_hbm.at[pl.ds((row0 + i + 4) * CHUNK, CHUNK)],
                    sbufs[b4], semi[b4])
                pltpu.async_copy(
                    dst_hbm.at[pl.ds((row0 + i + 4) * CHUNK, CHUNK)],
                    dbufs[b4], semi[b4])
            if b4 < 2:
                _idx_wait(src_hbm, sbufs[s2], dbufs[s2], semi[s2])
                pltpu.async_copy(ht_hbm.at[sbufs[s2]], gbufs[gi],
                                 semg[gi])
            else:
                @pl.when(g < nsteps - 1)
                def _():
                    _idx_wait(src_hbm, sbufs[s2], dbufs[s2], semi[s2])
                    pltpu.async_copy(ht_hbm.at[sbufs[s2]], gbufs[gi],
                                     semg[gi])
        return 0
    lax.fori_loop(0, nsteps, step, 0)


def _agg_epilogue(acc_sh, out_hbm, s):
    rows = R // NS
    pltpu.sync_copy(acc_sh.at[pl.ds(s * rows, rows)],
                    out_hbm.at[pl.ds(s * rows, rows)])


def _agg1_body(htl_hbm, htr_hbm, src_hbm, dst_hbm, aggl_hbm, aggr_hbm,
               gb0, gb1, sb0, sb1, sb2, sb3, db0, db1, db2, db3, acc_sh,
               smg0, smg1, smi0, smi1, smi2, smi3):
    c = lax.axis_index("c")
    s = lax.axis_index("s")
    nrows = EROWS // NS
    gbufs = (gb0, gb1)
    sbufs = (sb0, sb1, sb2, sb3)
    dbufs = (db0, db1, db2, db3)
    semg = (smg0, smg1)
    semi = (smi0, smi1, smi2, smi3)

    @pl.when(c == 0)
    def _():
        _agg_body_common(htl_hbm, src_hbm, dst_hbm, acc_sh, gbufs, sbufs,
                         dbufs, semg, semi, s, s * nrows, nrows)

    @pl.when(c == 1)
    def _():
        _agg_body_common(htr_hbm, src_hbm, dst_hbm, acc_sh, gbufs, sbufs,
                         dbufs, semg, semi, s, s * nrows, nrows)

    plsc.subcore_barrier()

    @pl.when(c == 0)
    def _():
        _agg_epilogue(acc_sh, aggl_hbm, s)

    @pl.when(c == 1)
    def _():
        _agg_epilogue(acc_sh, aggr_hbm, s)


def _agg_scratch(dsc):
    return [
        pltpu.VMEM((CHUNK, dsc), F32),
        pltpu.VMEM((CHUNK, dsc), F32),
        pltpu.VMEM((CHUNK,), jnp.int32),
        pltpu.VMEM((CHUNK,), jnp.int32),
        pltpu.VMEM((CHUNK,), jnp.int32),
        pltpu.VMEM((CHUNK,), jnp.int32),
        pltpu.VMEM((CHUNK,), jnp.int32),
        pltpu.VMEM((CHUNK,), jnp.int32),
        pltpu.VMEM((CHUNK,), jnp.int32),
        pltpu.VMEM((CHUNK,), jnp.int32),
        pltpu.VMEM_SHARED((R, dsc), F32),
        pltpu.SemaphoreType.DMA,
        pltpu.SemaphoreType.DMA,
        pltpu.SemaphoreType.DMA,
        pltpu.SemaphoreType.DMA,
        pltpu.SemaphoreType.DMA,
        pltpu.SemaphoreType.DMA,
    ]


_agg1_call = pl.kernel(
    _agg1_body,
    name='agg1k',
    out_type=[jax.ShapeDtypeStruct((R, D_HID // 2), F32),
              jax.ShapeDtypeStruct((R, D_HID // 2), F32)],
    mesh=_MESH,
    scratch_types=_agg_scratch(D_HID // 2),
)


def _agg2_body(ht_hbm, src_hbm, dst_hbm, agg0_hbm, agg1_hbm,
               gb0, gb1, sb0, sb1, sb2, sb3, db0, db1, db2, db3, acc_sh,
               smg0, smg1, smi0, smi1, smi2, smi3):
    c = lax.axis_index("c")
    s = lax.axis_index("s")
    nrows = EROWS // (NC * NS)
    _agg_body_common(ht_hbm, src_hbm, dst_hbm, acc_sh,
                     (gb0, gb1), (sb0, sb1, sb2, sb3),
                     (db0, db1, db2, db3), (smg0, smg1),
                     (smi0, smi1, smi2, smi3), s,
                     (c * NS + s) * nrows, nrows)
    plsc.subcore_barrier()

    @pl.when(c == 0)
    def _():
        _agg_epilogue(acc_sh, agg0_hbm, s)

    @pl.when(c == 1)
    def _():
        _agg_epilogue(acc_sh, agg1_hbm, s)


_agg2_call = pl.kernel(
    _agg2_body,
    name='agg2k',
    out_type=[jax.ShapeDtypeStruct((R, D_OUT), F32),
              jax.ShapeDtypeStruct((R, D_OUT), F32)],
    mesh=_MESH,
    scratch_types=_agg_scratch(D_OUT),
)


_BR = 256
_GRID = R // _BR


def _k1_body(x_ref, w1_ref, d0_ref, d1_ref, htl_ref, htr_ref, dis_ref):
    deg = d0_ref[...] + d1_ref[...] + 1.0
    dis = lax.rsqrt(deg)
    dis_ref[...] = dis
    h = jnp.dot(x_ref[...], w1_ref[...], preferred_element_type=F32)
    ht = h * dis
    htl_ref[...] = ht[:, :D_HID // 2]
    htr_ref[...] = ht[:, D_HID // 2:]


_k1_call = pl.pallas_call(
    _k1_body,
    grid=(_GRID,),
    in_specs=[
        pl.BlockSpec((_BR, D_IN), lambda i: (i, 0)),
        pl.BlockSpec((D_IN, D_HID), lambda i: (0, 0)),
        pl.BlockSpec((_BR, 1), lambda i: (i, 0)),
        pl.BlockSpec((_BR, 1), lambda i: (i, 0)),
    ],
    out_specs=[
        pl.BlockSpec((_BR, D_HID // 2), lambda i: (i, 0)),
        pl.BlockSpec((_BR, D_HID // 2), lambda i: (i, 0)),
        pl.BlockSpec((_BR, 1), lambda i: (i, 0)),
    ],
    out_shape=[
        jax.ShapeDtypeStruct((R, D_HID // 2), F32),
        jax.ShapeDtypeStruct((R, D_HID // 2), F32),
        jax.ShapeDtypeStruct((R, 1), F32),
    ],
)


def _k2_body(al_ref, ar_ref, hl_ref, hr_ref, dis_ref, b1_ref, w2_ref,
             hrelu_ref, ht2_ref):
    dis = dis_ref[...]
    s1 = (al_ref[...] + hl_ref[...]) * dis
    s2 = (ar_ref[...] + hr_ref[...]) * dis
    h1 = jnp.concatenate([s1, s2], axis=1) + b1_ref[...]
    hr = jnp.maximum(h1, 0.0)
    hrelu_ref[...] = hr
    h2 = jnp.dot(hr, w2_ref[...], preferred_element_type=F32)
    ht2_ref[...] = h2 * dis


_k2_call = pl.pallas_call(
    _k2_body,
    grid=(_GRID,),
    in_specs=[
        pl.BlockSpec((_BR, D_HID // 2), lambda i: (i, 0)),
        pl.BlockSpec((_BR, D_HID // 2), lambda i: (i, 0)),
        pl.BlockSpec((_BR, D_HID // 2), lambda i: (i, 0)),
        pl.BlockSpec((_BR, D_HID // 2), lambda i: (i, 0)),
        pl.BlockSpec((_BR, 1), lambda i: (i, 0)),
        pl.BlockSpec((1, D_HID), lambda i: (0, 0)),
        pl.BlockSpec((D_HID, D_OUT), lambda i: (0, 0)),
    ],
    out_specs=[
        pl.BlockSpec((_BR, D_HID), lambda i: (i, 0)),
        pl.BlockSpec((_BR, D_OUT), lambda i: (i, 0)),
    ],
    out_shape=[
        jax.ShapeDtypeStruct((R, D_HID), F32),
        jax.ShapeDtypeStruct((R, D_OUT), F32),
    ],
)


def _k3_body(a0_ref, a1_ref, ht2_ref, dis_ref, b2_ref, out_ref):
    s = a0_ref[...] + a1_ref[...] + ht2_ref[...]
    out_ref[...] = s * dis_ref[...] + b2_ref[...]


_k3_call = pl.pallas_call(
    _k3_body,
    grid=(_GRID,),
    in_specs=[
        pl.BlockSpec((_BR, D_OUT), lambda i: (i, 0)),
        pl.BlockSpec((_BR, D_OUT), lambda i: (i, 0)),
        pl.BlockSpec((_BR, D_OUT), lambda i: (i, 0)),
        pl.BlockSpec((_BR, 1), lambda i: (i, 0)),
        pl.BlockSpec((1, D_OUT), lambda i: (0, 0)),
    ],
    out_specs=pl.BlockSpec((_BR, D_OUT), lambda i: (i, 0)),
    out_shape=jax.ShapeDtypeStruct((R, D_OUT), F32),
)


def kernel(x, edge_index, W1, b1, W2, b2):
    pad_e = EPAD - E
    spread = (N + jnp.arange(pad_e, dtype=edge_index.dtype) % (R - N))
    src = jnp.concatenate([edge_index[0], spread])
    dst = jnp.concatenate([edge_index[1], spread])
    src2 = src.reshape(EROWS, CHUNK)
    dst2 = dst.reshape(EROWS, CHUNK)
    x_pad = jnp.zeros((R, D_IN), F32).at[:N].set(x)

    deg0, deg1 = _deg_call(dst2)
    htl, htr, dis = _k1_call(x_pad, W1, deg0.reshape(R, 1),
                             deg1.reshape(R, 1))
    a1l, a1r = _agg1_call(htl, htr, src, dst)
    hrelu, ht2 = _k2_call(a1l, a1r, htl, htr, dis, b1.reshape(1, -1), W2)
    a20, a21 = _agg2_call(ht2, src, dst)
    out2 = _k3_call(a20, a21, ht2, dis, b2.reshape(1, -1))
    return out2[:N], hrelu[:N]


# speedup vs baseline: 23.4672x; 1.0939x over previous
import jax
import jax.numpy as jnp
from jax import lax
from jax.experimental import pallas as pl
from jax.experimental.pallas import tpu as pltpu
from jax.experimental.pallas import tpu_sc as plsc

N = 10000
D_IN = 128
D_HID = 256
D_OUT = 128
E = 320000
R = 10240
CHUNK = 128
NC, NS = 2, 16
EROWS = 2560
EPAD = EROWS * CHUNK
NBUF = 4
HALF = R // 2
F32 = jnp.float32

_MESH = plsc.VectorSubcoreMesh(core_axis_name="c", subcore_axis_name="s")


def _deg_body(dst_hbm, deg0_hbm, deg1_hbm, idx_v, acc_v, red_v, tmp_v,
              stage_sh):
    c = lax.axis_index("c")
    s = lax.axis_index("s")
    nrows = EROWS // (NC * NS)
    lanes = lax.iota(jnp.int32, 16)
    lane_off = lanes * HALF
    ones = jnp.ones((16,), F32)
    pltpu.sync_copy(dst_hbm.at[pl.ds((c * NS + s) * nrows, nrows)], idx_v)

    for p in range(2):
        lo = p * HALF

        def zero_acc(i, _):
            for u in range(8):
                acc_v[pl.ds((i * 8 + u) * 16, 16)] = jnp.zeros((16,), F32)
            return 0
        lax.fori_loop(0, (16 * HALF) // 128, zero_acc, 0)

        def row_body(i, _):
            for k in range(CHUNK // 16):
                v = idx_v[i, pl.ds(k * 16, 16)]
                rel = v - lo
                m = (rel >= 0) & (rel < HALF)
                rel_c = jnp.clip(rel, 0, HALF - 1)
                plsc.addupdate_scatter(acc_v, [rel_c + lane_off], ones,
                                       mask=m)
            return 0
        lax.fori_loop(0, nrows, row_body, 0)

        def red_body(j, _):
            t = acc_v[pl.ds(j * 16, 16)]
            for l in range(1, 16):
                t = t + acc_v[pl.ds(l * HALF + j * 16, 16)]
            red_v[pl.ds(lo + j * 16, 16)] = t
            return 0
        lax.fori_loop(0, HALF // 16, red_body, 0)

    pltpu.sync_copy(red_v, stage_sh.at[s])
    plsc.subcore_barrier()
    rows = R // NS
    pltpu.sync_copy(stage_sh.at[:, pl.ds(s * rows, rows)], tmp_v)

    def add_body(j, _):
        acc = tmp_v[0, pl.ds(j * 16, 16)]
        for l in range(1, NS):
            acc = acc + tmp_v[l, pl.ds(j * 16, 16)]
        red_v[pl.ds(j * 16, 16)] = acc
        return 0
    lax.fori_loop(0, rows // 16, add_body, 0)

    @pl.when(c == 0)
    def _():
        pltpu.sync_copy(red_v.at[pl.ds(0, rows)],
                        deg0_hbm.at[pl.ds(s * rows, rows)])

    @pl.when(c == 1)
    def _():
        pltpu.sync_copy(red_v.at[pl.ds(0, rows)],
                        deg1_hbm.at[pl.ds(s * rows, rows)])


_deg_call = pl.kernel(
    _deg_body,
    name='degk',
    out_type=[jax.ShapeDtypeStruct((R,), F32),
              jax.ShapeDtypeStruct((R,), F32)],
    mesh=_MESH,
    compiler_params=pltpu.CompilerParams(needs_layout_passes=False),
    scratch_types=[
        pltpu.VMEM((EROWS // (NC * NS), CHUNK), jnp.int32),
        pltpu.VMEM((16 * HALF,), F32),
        pltpu.VMEM((R,), F32),
        pltpu.VMEM((NS, R // NS), F32),
        pltpu.VMEM_SHARED((NS, R), F32),
    ],
)


def _fill_zeros2d(ref, rows, cols):
    def body(i, _):
        for j in range(cols // 16):
            ref[i, pl.ds(j * 16, 16)] = jnp.zeros((16,), F32)
        return 0
    lax.fori_loop(0, rows, body, 0)


def _idx_wait(src_hbm, sb, db, semi):
    pltpu.make_async_copy(src_hbm.at[pl.ds(0, CHUNK)], sb, semi).wait()
    pltpu.make_async_copy(src_hbm.at[pl.ds(0, CHUNK)], db, semi).wait()


def _agg_body_common(ht_hbm, src_hbm, dst_hbm, acc_sh, gbufs, sbufs, dbufs,
                     semg, semi, s, row0, nrows):
    _fill_zeros2d(gbufs[0], CHUNK, gbufs[0].shape[1])
    rows = R // NS
    zdescs = [pltpu.make_async_copy(
        gbufs[0], acc_sh.at[pl.ds(s * rows + k * CHUNK, CHUNK)], semg[0])
        for k in range(rows // CHUNK)]
    for d in zdescs:
        d.start()
    for d in zdescs:
        d.wait()
    plsc.subcore_barrier()

    for tslot in range(4):
        pltpu.async_copy(src_hbm.at[pl.ds((row0 + tslot) * CHUNK, CHUNK)],
                         sbufs[tslot], semi[tslot])
        pltpu.async_copy(dst_hbm.at[pl.ds((row0 + tslot) * CHUNK, CHUNK)],
                         dbufs[tslot], semi[tslot])
    for bg in range(2):
        _idx_wait(src_hbm, sbufs[bg], dbufs[bg], semi[bg])
        pltpu.async_copy(ht_hbm.at[sbufs[bg]], gbufs[bg], semg[bg])

    nsteps = nrows // 4

    def step(g, _):
        for b4 in range(4):
            i = g * 4 + b4
            gi = b4 % 2
            s2 = (b4 + 2) % 4
            pltpu.make_async_copy(ht_hbm.at[sbufs[b4]], gbufs[gi],
                                  semg[gi]).wait()
            pltpu.sync_copy(gbufs[gi], acc_sh.at[dbufs[b4]], add=True)
            @pl.when(g < nsteps - 1)
            def _():
                pltpu.async_copy(
                    src_hbm.at[pl.ds((row0 + i + 4) * CHUNK, CHUNK)],
                    sbufs[b4], semi[b4])
                pltpu.async_copy(
                    dst_hbm.at[pl.ds((row0 + i + 4) * CHUNK, CHUNK)],
                    dbufs[b4], semi[b4])
            if b4 < 2:
                _idx_wait(src_hbm, sbufs[s2], dbufs[s2], semi[s2])
                pltpu.async_copy(ht_hbm.at[sbufs[s2]], gbufs[gi],
                                 semg[gi])
            else:
                @pl.when(g < nsteps - 1)
                def _():
                    _idx_wait(src_hbm, sbufs[s2], dbufs[s2], semi[s2])
                    pltpu.async_copy(ht_hbm.at[sbufs[s2]], gbufs[gi],
                                     semg[gi])
        return 0
    lax.fori_loop(0, nsteps, step, 0)


def _agg_epilogue(acc_sh, out_hbm, s):
    rows = R // NS
    pltpu.sync_copy(acc_sh.at[pl.ds(s * rows, rows)],
                    out_hbm.at[pl.ds(s * rows, rows)])


def _agg1_body(htl_hbm, htr_hbm, src_hbm, dst_hbm, aggl_hbm, aggr_hbm,
               gb0, gb1, sb0, sb1, sb2, sb3, db0, db1, db2, db3, acc_sh,
               smg0, smg1, smi0, smi1, smi2, smi3):
    c = lax.axis_index("c")
    s = lax.axis_index("s")
    nrows = EROWS // NS
    gbufs = (gb0, gb1)
    sbufs = (sb0, sb1, sb2, sb3)
    dbufs = (db0, db1, db2, db3)
    semg = (smg0, smg1)
    semi = (smi0, smi1, smi2, smi3)

    @pl.when(c == 0)
    def _():
        _agg_body_common(htl_hbm, src_hbm, dst_hbm, acc_sh, gbufs, sbufs,
                         dbufs, semg, semi, s, s * nrows, nrows)

    @pl.when(c == 1)
    def _():
        _agg_body_common(htr_hbm, src_hbm, dst_hbm, acc_sh, gbufs, sbufs,
                         dbufs, semg, semi, s, s * nrows, nrows)

    plsc.subcore_barrier()

    @pl.when(c == 0)
    def _():
        _agg_epilogue(acc_sh, aggl_hbm, s)

    @pl.when(c == 1)
    def _():
        _agg_epilogue(acc_sh, aggr_hbm, s)


def _agg_scratch(dsc):
    return [
        pltpu.VMEM((CHUNK, dsc), F32),
        pltpu.VMEM((CHUNK, dsc), F32),
        pltpu.VMEM((CHUNK,), jnp.int32),
        pltpu.VMEM((CHUNK,), jnp.int32),
        pltpu.VMEM((CHUNK,), jnp.int32),
        pltpu.VMEM((CHUNK,), jnp.int32),
        pltpu.VMEM((CHUNK,), jnp.int32),
        pltpu.VMEM((CHUNK,), jnp.int32),
        pltpu.VMEM((CHUNK,), jnp.int32),
        pltpu.VMEM((CHUNK,), jnp.int32),
        pltpu.VMEM_SHARED((R, dsc), F32),
        pltpu.SemaphoreType.DMA,
        pltpu.SemaphoreType.DMA,
        pltpu.SemaphoreType.DMA,
        pltpu.SemaphoreType.DMA,
        pltpu.SemaphoreType.DMA,
        pltpu.SemaphoreType.DMA,
    ]


_agg1_call = pl.kernel(
    _agg1_body,
    name='agg1k',
    out_type=[jax.ShapeDtypeStruct((R, D_HID // 2), F32),
              jax.ShapeDtypeStruct((R, D_HID // 2), F32)],
    mesh=_MESH,
    scratch_types=_agg_scratch(D_HID // 2),
)


def _agg2_body(ht_hbm, src_hbm, dst_hbm, agg0_hbm, agg1_hbm,
               gb0, gb1, sb0, sb1, sb2, sb3, db0, db1, db2, db3, acc_sh,
               smg0, smg1, smi0, smi1, smi2, smi3):
    c = lax.axis_index("c")
    s = lax.axis_index("s")
    nrows = EROWS // (NC * NS)
    _agg_body_common(ht_hbm, src_hbm, dst_hbm, acc_sh,
                     (gb0, gb1), (sb0, sb1, sb2, sb3),
                     (db0, db1, db2, db3), (smg0, smg1),
                     (smi0, smi1, smi2, smi3), s,
                     (c * NS + s) * nrows, nrows)
    plsc.subcore_barrier()

    @pl.when(c == 0)
    def _():
        _agg_epilogue(acc_sh, agg0_hbm, s)

    @pl.when(c == 1)
    def _():
        _agg_epilogue(acc_sh, agg1_hbm, s)


_agg2_call = pl.kernel(
    _agg2_body,
    name='agg2k',
    out_type=[jax.ShapeDtypeStruct((R, D_OUT), F32),
              jax.ShapeDtypeStruct((R, D_OUT), F32)],
    mesh=_MESH,
    scratch_types=_agg_scratch(D_OUT),
)


_BR = 256
_GRID = R // _BR


def _k1a_body(x_ref, w1_ref, h_ref):
    h_ref[...] = jnp.dot(x_ref[...], w1_ref[...],
                         preferred_element_type=F32)


_k1a_call = pl.pallas_call(
    _k1a_body,
    grid=(_GRID,),
    in_specs=[
        pl.BlockSpec((_BR, D_IN), lambda i: (i, 0)),
        pl.BlockSpec((D_IN, D_HID), lambda i: (0, 0)),
    ],
    out_specs=pl.BlockSpec((_BR, D_HID), lambda i: (i, 0)),
    out_shape=jax.ShapeDtypeStruct((R, D_HID), F32),
)


def _k1b_body(h_ref, d0_ref, d1_ref, htl_ref, htr_ref, dis_ref):
    deg = d0_ref[...] + d1_ref[...] + 1.0
    dis = lax.rsqrt(deg)
    dis_ref[...] = dis
    ht = h_ref[...] * dis
    htl_ref[...] = ht[:, :D_HID // 2]
    htr_ref[...] = ht[:, D_HID // 2:]


_k1b_call = pl.pallas_call(
    _k1b_body,
    grid=(_GRID,),
    in_specs=[
        pl.BlockSpec((_BR, D_HID), lambda i: (i, 0)),
        pl.BlockSpec((_BR, 1), lambda i: (i, 0)),
        pl.BlockSpec((_BR, 1), lambda i: (i, 0)),
    ],
    out_specs=[
        pl.BlockSpec((_BR, D_HID // 2), lambda i: (i, 0)),
        pl.BlockSpec((_BR, D_HID // 2), lambda i: (i, 0)),
        pl.BlockSpec((_BR, 1), lambda i: (i, 0)),
    ],
    out_shape=[
        jax.ShapeDtypeStruct((R, D_HID // 2), F32),
        jax.ShapeDtypeStruct((R, D_HID // 2), F32),
        jax.ShapeDtypeStruct((R, 1), F32),
    ],
)


def _k2_body(al_ref, ar_ref, hl_ref, hr_ref, dis_ref, b1_ref, w2_ref,
             hrelu_ref, ht2_ref):
    dis = dis_ref[...]
    s1 = (al_ref[...] + hl_ref[...]) * dis
    s2 = (ar_ref[...] + hr_ref[...]) * dis
    h1 = jnp.concatenate([s1, s2], axis=1) + b1_ref[...]
    hr = jnp.maximum(h1, 0.0)
    hrelu_ref[...] = hr
    h2 = jnp.dot(hr, w2_ref[...], preferred_element_type=F32)
    ht2_ref[...] = h2 * dis


_k2_call = pl.pallas_call(
    _k2_body,
    grid=(_GRID,),
    in_specs=[
        pl.BlockSpec((_BR, D_HID // 2), lambda i: (i, 0)),
        pl.BlockSpec((_BR, D_HID // 2), lambda i: (i, 0)),
        pl.BlockSpec((_BR, D_HID // 2), lambda i: (i, 0)),
        pl.BlockSpec((_BR, D_HID // 2), lambda i: (i, 0)),
        pl.BlockSpec((_BR, 1), lambda i: (i, 0)),
        pl.BlockSpec((1, D_HID), lambda i: (0, 0)),
        pl.BlockSpec((D_HID, D_OUT), lambda i: (0, 0)),
    ],
    out_specs=[
        pl.BlockSpec((_BR, D_HID), lambda i: (i, 0)),
        pl.BlockSpec((_BR, D_OUT), lambda i: (i, 0)),
    ],
    out_shape=[
        jax.ShapeDtypeStruct((R, D_HID), F32),
        jax.ShapeDtypeStruct((R, D_OUT), F32),
    ],
)


def _k3_body(a0_ref, a1_ref, ht2_ref, dis_ref, b2_ref, out_ref):
    s = a0_ref[...] + a1_ref[...] + ht2_ref[...]
    out_ref[...] = s * dis_ref[...] + b2_ref[...]


_k3_call = pl.pallas_call(
    _k3_body,
    grid=(_GRID,),
    in_specs=[
        pl.BlockSpec((_BR, D_OUT), lambda i: (i, 0)),
        pl.BlockSpec((_BR, D_OUT), lambda i: (i, 0)),
        pl.BlockSpec((_BR, D_OUT), lambda i: (i, 0)),
        pl.BlockSpec((_BR, 1), lambda i: (i, 0)),
        pl.BlockSpec((1, D_OUT), lambda i: (0, 0)),
    ],
    out_specs=pl.BlockSpec((_BR, D_OUT), lambda i: (i, 0)),
    out_shape=jax.ShapeDtypeStruct((R, D_OUT), F32),
)


def kernel(x, edge_index, W1, b1, W2, b2):
    pad_e = EPAD - E
    spread = (N + jnp.arange(pad_e, dtype=edge_index.dtype) % (R - N))
    src = jnp.concatenate([edge_index[0], spread])
    dst = jnp.concatenate([edge_index[1], spread])
    src2 = src.reshape(EROWS, CHUNK)
    dst2 = dst.reshape(EROWS, CHUNK)
    x_pad = jnp.zeros((R, D_IN), F32).at[:N].set(x)

    deg0, deg1 = _deg_call(dst2)
    h1 = _k1a_call(x_pad, W1)
    htl, htr, dis = _k1b_call(h1, deg0.reshape(R, 1), deg1.reshape(R, 1))
    a1l, a1r = _agg1_call(htl, htr, src, dst)
    hrelu, ht2 = _k2_call(a1l, a1r, htl, htr, dis, b1.reshape(1, -1), W2)
    a20, a21 = _agg2_call(ht2, src, dst)
    out2 = _k3_call(a20, a21, ht2, dis, b2.reshape(1, -1))
    return out2[:N], hrelu[:N]


# speedup vs baseline: 29.1198x; 1.2409x over previous
import jax
import jax.numpy as jnp
from jax import lax
from jax.experimental import pallas as pl
from jax.experimental.pallas import tpu as pltpu
from jax.experimental.pallas import tpu_sc as plsc

N = 10000
D_IN = 128
D_HID = 256
D_OUT = 128
E = 320000
R = 10240
CHUNK = 128
NC, NS = 2, 16
EROWS = 2560
EPAD = EROWS * CHUNK
NBUF = 4
HALF = R // 2
F32 = jnp.float32

_MESH = plsc.VectorSubcoreMesh(core_axis_name="c", subcore_axis_name="s")


def _deg_body(dst_hbm, deg0_hbm, deg1_hbm, idx_v, acc_v, red_v, tmp_v,
              stage_sh):
    c = lax.axis_index("c")
    s = lax.axis_index("s")
    nrows = EROWS // (NC * NS)
    lanes = lax.iota(jnp.int32, 16)
    lane_off = lanes * HALF
    ones = jnp.ones((16,), F32)
    pltpu.sync_copy(dst_hbm.at[pl.ds((c * NS + s) * nrows, nrows)], idx_v)

    for p in range(2):
        lo = p * HALF

        def zero_acc(i, _):
            for u in range(8):
                acc_v[pl.ds((i * 8 + u) * 16, 16)] = jnp.zeros((16,), F32)
            return 0
        lax.fori_loop(0, (16 * HALF) // 128, zero_acc, 0)

        def row_body(i, _):
            for k in range(CHUNK // 16):
                v = idx_v[i, pl.ds(k * 16, 16)]
                rel = v - lo
                m = (rel >= 0) & (rel < HALF)
                rel_c = jnp.clip(rel, 0, HALF - 1)
                plsc.addupdate_scatter(acc_v, [rel_c + lane_off], ones,
                                       mask=m)
            return 0
        lax.fori_loop(0, nrows, row_body, 0)

        def red_body(j, _):
            t = acc_v[pl.ds(j * 16, 16)]
            for l in range(1, 16):
                t = t + acc_v[pl.ds(l * HALF + j * 16, 16)]
            red_v[pl.ds(lo + j * 16, 16)] = t
            return 0
        lax.fori_loop(0, HALF // 16, red_body, 0)

    pltpu.sync_copy(red_v, stage_sh.at[s])
    plsc.subcore_barrier()
    rows = R // NS
    pltpu.sync_copy(stage_sh.at[:, pl.ds(s * rows, rows)], tmp_v)

    def add_body(j, _):
        acc = tmp_v[0, pl.ds(j * 16, 16)]
        for l in range(1, NS):
            acc = acc + tmp_v[l, pl.ds(j * 16, 16)]
        red_v[pl.ds(j * 16, 16)] = acc
        return 0
    lax.fori_loop(0, rows // 16, add_body, 0)

    @pl.when(c == 0)
    def _():
        pltpu.sync_copy(red_v.at[pl.ds(0, rows)],
                        deg0_hbm.at[pl.ds(s * rows, rows)])

    @pl.when(c == 1)
    def _():
        pltpu.sync_copy(red_v.at[pl.ds(0, rows)],
                        deg1_hbm.at[pl.ds(s * rows, rows)])


_deg_call = pl.kernel(
    _deg_body,
    name='degk',
    out_type=[jax.ShapeDtypeStruct((R,), F32),
              jax.ShapeDtypeStruct((R,), F32)],
    mesh=_MESH,
    compiler_params=pltpu.CompilerParams(needs_layout_passes=False),
    scratch_types=[
        pltpu.VMEM((EROWS // (NC * NS), CHUNK), jnp.int32),
        pltpu.VMEM((16 * HALF,), F32),
        pltpu.VMEM((R,), F32),
        pltpu.VMEM((NS, R // NS), F32),
        pltpu.VMEM_SHARED((NS, R), F32),
    ],
)


def _fill_zeros2d(ref, rows, cols):
    def body(i, _):
        for j in range(cols // 16):
            ref[i, pl.ds(j * 16, 16)] = jnp.zeros((16,), F32)
        return 0
    lax.fori_loop(0, rows, body, 0)


def _idx_wait(src_hbm, sb, db, semi):
    pltpu.make_async_copy(src_hbm.at[pl.ds(0, CHUNK)], sb, semi).wait()
    pltpu.make_async_copy(src_hbm.at[pl.ds(0, CHUNK)], db, semi).wait()


def _agg_body_common(ht_hbm, src_hbm, dst_hbm, acc_sh, gbufs, sbufs, dbufs,
                     semg, semi, s, row0, nrows):
    _fill_zeros2d(gbufs[0], CHUNK, gbufs[0].shape[1])
    rows = R // NS
    zdescs = [pltpu.make_async_copy(
        gbufs[0], acc_sh.at[pl.ds(s * rows + k * CHUNK, CHUNK)], semg[0])
        for k in range(rows // CHUNK)]
    for d in zdescs:
        d.start()
    for d in zdescs:
        d.wait()
    plsc.subcore_barrier()

    for tslot in range(4):
        pltpu.async_copy(src_hbm.at[pl.ds((row0 + tslot) * CHUNK, CHUNK)],
                         sbufs[tslot], semi[tslot])
        pltpu.async_copy(dst_hbm.at[pl.ds((row0 + tslot) * CHUNK, CHUNK)],
                         dbufs[tslot], semi[tslot])
    for bg in range(2):
        _idx_wait(src_hbm, sbufs[bg], dbufs[bg], semi[bg])
        pltpu.async_copy(ht_hbm.at[sbufs[bg]], gbufs[bg], semg[bg])

    nsteps = nrows // 4

    def step(g, _):
        for b4 in range(4):
            i = g * 4 + b4
            gi = b4 % 2
            s2 = (b4 + 2) % 4
            pltpu.make_async_copy(ht_hbm.at[sbufs[b4]], gbufs[gi],
                                  semg[gi]).wait()
            pltpu.sync_copy(gbufs[gi], acc_sh.at[dbufs[b4]], add=True)
            @pl.when(g < nsteps - 1)
            def _():
                pltpu.async_copy(
                    src_hbm.at[pl.ds((row0 + i + 4) * CHUNK, CHUNK)],
                    sbufs[b4], semi[b4])
                pltpu.async_copy(
                    dst_hbm.at[pl.ds((row0 + i + 4) * CHUNK, CHUNK)],
                    dbufs[b4], semi[b4])
            if b4 < 2:
                _idx_wait(src_hbm, sbufs[s2], dbufs[s2], semi[s2])
                pltpu.async_copy(ht_hbm.at[sbufs[s2]], gbufs[gi],
                                 semg[gi])
            else:
                @pl.when(g < nsteps - 1)
                def _():
                    _idx_wait(src_hbm, sbufs[s2], dbufs[s2], semi[s2])
                    pltpu.async_copy(ht_hbm.at[sbufs[s2]], gbufs[gi],
                                     semg[gi])
        return 0
    lax.fori_loop(0, nsteps, step, 0)


def _agg_epilogue(acc_sh, out_hbm, s):
    rows = R // NS
    pltpu.sync_copy(acc_sh.at[pl.ds(s * rows, rows)],
                    out_hbm.at[pl.ds(s * rows, rows)])


def _agg_scratch(dsc):
    return [
        pltpu.VMEM((CHUNK, dsc), F32),
        pltpu.VMEM((CHUNK, dsc), F32),
        pltpu.VMEM((CHUNK,), jnp.int32),
        pltpu.VMEM((CHUNK,), jnp.int32),
        pltpu.VMEM((CHUNK,), jnp.int32),
        pltpu.VMEM((CHUNK,), jnp.int32),
        pltpu.VMEM((CHUNK,), jnp.int32),
        pltpu.VMEM((CHUNK,), jnp.int32),
        pltpu.VMEM((CHUNK,), jnp.int32),
        pltpu.VMEM((CHUNK,), jnp.int32),
        pltpu.VMEM_SHARED((R, dsc), F32),
        pltpu.SemaphoreType.DMA,
        pltpu.SemaphoreType.DMA,
        pltpu.SemaphoreType.DMA,
        pltpu.SemaphoreType.DMA,
        pltpu.SemaphoreType.DMA,
        pltpu.SemaphoreType.DMA,
    ]


def _agg2_body(ht_hbm, src_hbm, dst_hbm, agg0_hbm, agg1_hbm,
               gb0, gb1, sb0, sb1, sb2, sb3, db0, db1, db2, db3, acc_sh,
               smg0, smg1, smi0, smi1, smi2, smi3):
    c = lax.axis_index("c")
    s = lax.axis_index("s")
    nrows = EROWS // (NC * NS)
    _agg_body_common(ht_hbm, src_hbm, dst_hbm, acc_sh,
                     (gb0, gb1), (sb0, sb1, sb2, sb3),
                     (db0, db1, db2, db3), (smg0, smg1),
                     (smi0, smi1, smi2, smi3), s,
                     (c * NS + s) * nrows, nrows)
    plsc.subcore_barrier()

    @pl.when(c == 0)
    def _():
        _agg_epilogue(acc_sh, agg0_hbm, s)

    @pl.when(c == 1)
    def _():
        _agg_epilogue(acc_sh, agg1_hbm, s)


_agg2_call = pl.kernel(
    _agg2_body,
    name='agg2k',
    out_type=[jax.ShapeDtypeStruct((R, D_OUT), F32),
              jax.ShapeDtypeStruct((R, D_OUT), F32)],
    mesh=_MESH,
    scratch_types=_agg_scratch(D_OUT),
)


_BR = 256
_GRID = R // _BR


def _s1_body(x_ref, d0_ref, d1_ref, xd_ref, dis_ref):
    deg = d0_ref[...] + d1_ref[...] + 1.0
    dis = lax.rsqrt(deg)
    dis_ref[...] = dis
    xd_ref[...] = x_ref[...] * dis


_s1_call = pl.pallas_call(
    _s1_body,
    grid=(_GRID,),
    in_specs=[
        pl.BlockSpec((_BR, D_IN), lambda i: (i, 0)),
        pl.BlockSpec((_BR, 1), lambda i: (i, 0)),
        pl.BlockSpec((_BR, 1), lambda i: (i, 0)),
    ],
    out_specs=[
        pl.BlockSpec((_BR, D_IN), lambda i: (i, 0)),
        pl.BlockSpec((_BR, 1), lambda i: (i, 0)),
    ],
    out_shape=[
        jax.ShapeDtypeStruct((R, D_IN), F32),
        jax.ShapeDtypeStruct((R, 1), F32),
    ],
)


def _t2_body(p0_ref, p1_ref, xd_ref, dis_ref, b1_ref, w1_ref, w2_ref,
             hrelu_ref, ht2_ref):
    dis = dis_ref[...]
    u = p0_ref[...] + p1_ref[...] + xd_ref[...]
    h1 = jnp.dot(u, w1_ref[...], preferred_element_type=F32)
    out1 = h1 * dis + b1_ref[...]
    hr = jnp.maximum(out1, 0.0)
    hrelu_ref[...] = hr
    ht2_ref[...] = jnp.dot(hr, w2_ref[...],
                           preferred_element_type=F32) * dis


_t2_call = pl.pallas_call(
    _t2_body,
    grid=(_GRID,),
    in_specs=[
        pl.BlockSpec((_BR, D_IN), lambda i: (i, 0)),
        pl.BlockSpec((_BR, D_IN), lambda i: (i, 0)),
        pl.BlockSpec((_BR, D_IN), lambda i: (i, 0)),
        pl.BlockSpec((_BR, 1), lambda i: (i, 0)),
        pl.BlockSpec((1, D_HID), lambda i: (0, 0)),
        pl.BlockSpec((D_IN, D_HID), lambda i: (0, 0)),
        pl.BlockSpec((D_HID, D_OUT), lambda i: (0, 0)),
    ],
    out_specs=[
        pl.BlockSpec((_BR, D_HID), lambda i: (i, 0)),
        pl.BlockSpec((_BR, D_OUT), lambda i: (i, 0)),
    ],
    out_shape=[
        jax.ShapeDtypeStruct((R, D_HID), F32),
        jax.ShapeDtypeStruct((R, D_OUT), F32),
    ],
)


def _k3_body(a0_ref, a1_ref, ht2_ref, dis_ref, b2_ref, out_ref):
    s = a0_ref[...] + a1_ref[...] + ht2_ref[...]
    out_ref[...] = s * dis_ref[...] + b2_ref[...]


_k3_call = pl.pallas_call(
    _k3_body,
    grid=(_GRID,),
    in_specs=[
        pl.BlockSpec((_BR, D_OUT), lambda i: (i, 0)),
        pl.BlockSpec((_BR, D_OUT), lambda i: (i, 0)),
        pl.BlockSpec((_BR, D_OUT), lambda i: (i, 0)),
        pl.BlockSpec((_BR, 1), lambda i: (i, 0)),
        pl.BlockSpec((1, D_OUT), lambda i: (0, 0)),
    ],
    out_specs=pl.BlockSpec((_BR, D_OUT), lambda i: (i, 0)),
    out_shape=jax.ShapeDtypeStruct((R, D_OUT), F32),
)


def kernel(x, edge_index, W1, b1, W2, b2):
    pad_e = EPAD - E
    spread = (N + jnp.arange(pad_e, dtype=edge_index.dtype) % (R - N))
    src = jnp.concatenate([edge_index[0], spread])
    dst = jnp.concatenate([edge_index[1], spread])
    src2 = src.reshape(EROWS, CHUNK)
    dst2 = dst.reshape(EROWS, CHUNK)
    x_pad = jnp.zeros((R, D_IN), F32).at[:N].set(x)

    deg0, deg1 = _deg_call(dst2)
    xd, dis = _s1_call(x_pad, deg0.reshape(R, 1), deg1.reshape(R, 1))
    p0, p1 = _agg2_call(xd, src, dst)
    hrelu, ht2 = _t2_call(p0, p1, xd, dis, b1.reshape(1, -1), W1, W2)
    q0, q1 = _agg2_call(ht2, src, dst)
    out2 = _k3_call(q0, q1, ht2, dis, b2.reshape(1, -1))
    return out2[:N], hrelu[:N]


# speedup vs baseline: 33.9602x; 1.1662x over previous
import jax
import jax.numpy as jnp
from jax import lax
from jax.experimental import pallas as pl
from jax.experimental.pallas import tpu as pltpu
from jax.experimental.pallas import tpu_sc as plsc

N = 10000
D_IN = 128
D_HID = 256
D_OUT = 128
E = 320000
R = 10240
CHUNK = 128
NC, NS = 2, 16
EROWS = 2560
EPAD = EROWS * CHUNK
NBUF = 4
HALF = R // 2
F32 = jnp.float32

_MESH = plsc.VectorSubcoreMesh(core_axis_name="c", subcore_axis_name="s")


def _deg_body(dst_hbm, deg0_hbm, deg1_hbm, idx_v, acc_v, red_v, tmp_v,
              stage_sh):
    c = lax.axis_index("c")
    s = lax.axis_index("s")
    nrows = EROWS // (NC * NS)
    lanes = lax.iota(jnp.int32, 16)
    lane_off = lanes * HALF
    ones = jnp.ones((16,), F32)
    pltpu.sync_copy(dst_hbm.at[pl.ds((c * NS + s) * nrows, nrows)], idx_v)

    for p in range(2):
        lo = p * HALF

        def zero_acc(i, _):
            for u in range(8):
                acc_v[pl.ds((i * 8 + u) * 16, 16)] = jnp.zeros((16,), F32)
            return 0
        lax.fori_loop(0, (16 * HALF) // 128, zero_acc, 0)

        def row_body(i, _):
            for k in range(CHUNK // 16):
                v = idx_v[i, pl.ds(k * 16, 16)]
                rel = v - lo
                m = (rel >= 0) & (rel < HALF)
                rel_c = jnp.clip(rel, 0, HALF - 1)
                plsc.addupdate_scatter(acc_v, [rel_c + lane_off], ones,
                                       mask=m)
            return 0
        lax.fori_loop(0, nrows, row_body, 0)

        def red_body(j, _):
            t = acc_v[pl.ds(j * 16, 16)]
            for l in range(1, 16):
                t = t + acc_v[pl.ds(l * HALF + j * 16, 16)]
            red_v[pl.ds(lo + j * 16, 16)] = t
            return 0
        lax.fori_loop(0, HALF // 16, red_body, 0)

    pltpu.sync_copy(red_v, stage_sh.at[s])
    plsc.subcore_barrier()
    rows = R // NS
    pltpu.sync_copy(stage_sh.at[:, pl.ds(s * rows, rows)], tmp_v)

    def add_body(j, _):
        acc = tmp_v[0, pl.ds(j * 16, 16)]
        for l in range(1, NS):
            acc = acc + tmp_v[l, pl.ds(j * 16, 16)]
        red_v[pl.ds(j * 16, 16)] = acc
        return 0
    lax.fori_loop(0, rows // 16, add_body, 0)

    @pl.when(c == 0)
    def _():
        pltpu.sync_copy(red_v.at[pl.ds(0, rows)],
                        deg0_hbm.at[pl.ds(s * rows, rows)])

    @pl.when(c == 1)
    def _():
        pltpu.sync_copy(red_v.at[pl.ds(0, rows)],
                        deg1_hbm.at[pl.ds(s * rows, rows)])


_deg_call = pl.kernel(
    _deg_body,
    name='degk',
    out_type=[jax.ShapeDtypeStruct((R,), F32),
              jax.ShapeDtypeStruct((R,), F32)],
    mesh=_MESH,
    compiler_params=pltpu.CompilerParams(needs_layout_passes=False),
    scratch_types=[
        pltpu.VMEM((EROWS // (NC * NS), CHUNK), jnp.int32),
        pltpu.VMEM((16 * HALF,), F32),
        pltpu.VMEM((R,), F32),
        pltpu.VMEM((NS, R // NS), F32),
        pltpu.VMEM_SHARED((NS, R), F32),
    ],
)


def _fill_zeros2d(ref, rows, cols):
    def body(i, _):
        for j in range(cols // 16):
            ref[i, pl.ds(j * 16, 16)] = jnp.zeros((16,), F32)
        return 0
    lax.fori_loop(0, rows, body, 0)


def _idx_wait(src_hbm, sb, db, semi):
    pltpu.make_async_copy(src_hbm.at[pl.ds(0, CHUNK)], sb, semi).wait()
    pltpu.make_async_copy(src_hbm.at[pl.ds(0, CHUNK)], db, semi).wait()


def _agg_body_common(ht_hbm, src_hbm, dst_hbm, acc_sh, gbufs, sbufs, dbufs,
                     semg, semi, s, row0, nrows):
    _fill_zeros2d(gbufs[0], CHUNK, gbufs[0].shape[1])
    rows = R // NS
    zdescs = [pltpu.make_async_copy(
        gbufs[0], acc_sh.at[pl.ds(s * rows + k * CHUNK, CHUNK)], semg[0])
        for k in range(rows // CHUNK)]
    for d in zdescs:
        d.start()
    for d in zdescs:
        d.wait()
    plsc.subcore_barrier()

    for tslot in range(4):
        pltpu.async_copy(src_hbm.at[pl.ds((row0 + tslot) * CHUNK, CHUNK)],
                         sbufs[tslot], semi[tslot])
        pltpu.async_copy(dst_hbm.at[pl.ds((row0 + tslot) * CHUNK, CHUNK)],
                         dbufs[tslot], semi[tslot])
    for bg in range(2):
        _idx_wait(src_hbm, sbufs[bg], dbufs[bg], semi[bg])
        pltpu.async_copy(ht_hbm.at[sbufs[bg]], gbufs[bg], semg[bg])

    nsteps = nrows // 4

    def step(g, _):
        for b4 in range(4):
            i = g * 4 + b4
            gi = b4 % 2
            s2 = (b4 + 2) % 4
            pltpu.make_async_copy(ht_hbm.at[sbufs[b4]], gbufs[gi],
                                  semg[gi]).wait()
            pltpu.sync_copy(gbufs[gi], acc_sh.at[dbufs[b4]], add=True)
            @pl.when(g < nsteps - 1)
            def _():
                pltpu.async_copy(
                    src_hbm.at[pl.ds((row0 + i + 4) * CHUNK, CHUNK)],
                    sbufs[b4], semi[b4])
                pltpu.async_copy(
                    dst_hbm.at[pl.ds((row0 + i + 4) * CHUNK, CHUNK)],
                    dbufs[b4], semi[b4])
            if b4 < 2:
                _idx_wait(src_hbm, sbufs[s2], dbufs[s2], semi[s2])
                pltpu.async_copy(ht_hbm.at[sbufs[s2]], gbufs[gi],
                                 semg[gi])
            else:
                @pl.when(g < nsteps - 1)
                def _():
                    _idx_wait(src_hbm, sbufs[s2], dbufs[s2], semi[s2])
                    pltpu.async_copy(ht_hbm.at[sbufs[s2]], gbufs[gi],
                                     semg[gi])
        return 0
    lax.fori_loop(0, nsteps, step, 0)


def _agg_epilogue(acc_sh, out_hbm, s):
    rows = R // NS
    pltpu.sync_copy(acc_sh.at[pl.ds(s * rows, rows)],
                    out_hbm.at[pl.ds(s * rows, rows)])


def _agg_scratch(dsc):
    return [
        pltpu.VMEM((CHUNK, dsc), F32),
        pltpu.VMEM((CHUNK, dsc), F32),
        pltpu.VMEM((CHUNK,), jnp.int32),
        pltpu.VMEM((CHUNK,), jnp.int32),
        pltpu.VMEM((CHUNK,), jnp.int32),
        pltpu.VMEM((CHUNK,), jnp.int32),
        pltpu.VMEM((CHUNK,), jnp.int32),
        pltpu.VMEM((CHUNK,), jnp.int32),
        pltpu.VMEM((CHUNK,), jnp.int32),
        pltpu.VMEM((CHUNK,), jnp.int32),
        pltpu.VMEM_SHARED((R, dsc), F32),
        pltpu.SemaphoreType.DMA,
        pltpu.SemaphoreType.DMA,
        pltpu.SemaphoreType.DMA,
        pltpu.SemaphoreType.DMA,
        pltpu.SemaphoreType.DMA,
        pltpu.SemaphoreType.DMA,
    ]


def _agg2_body(ht_hbm, src_hbm, dst_hbm, agg0_hbm, agg1_hbm,
               gb0, gb1, sb0, sb1, sb2, sb3, db0, db1, db2, db3, acc_sh,
               smg0, smg1, smi0, smi1, smi2, smi3):
    c = lax.axis_index("c")
    s = lax.axis_index("s")
    nrows = EROWS // (NC * NS)
    _agg_body_common(ht_hbm, src_hbm, dst_hbm, acc_sh,
                     (gb0, gb1), (sb0, sb1, sb2, sb3),
                     (db0, db1, db2, db3), (smg0, smg1),
                     (smi0, smi1, smi2, smi3), s,
                     (c * NS + s) * nrows, nrows)
    plsc.subcore_barrier()

    @pl.when(c == 0)
    def _():
        _agg_epilogue(acc_sh, agg0_hbm, s)

    @pl.when(c == 1)
    def _():
        _agg_epilogue(acc_sh, agg1_hbm, s)


_agg2_call = pl.kernel(
    _agg2_body,
    name='agg2k',
    out_type=[jax.ShapeDtypeStruct((R, D_OUT), F32),
              jax.ShapeDtypeStruct((R, D_OUT), F32)],
    mesh=_MESH,
    scratch_types=_agg_scratch(D_OUT),
)


_BR = 1024
_GRID = R // _BR


def _s1_body(x_ref, d0_ref, d1_ref, xd_ref, dis_ref):
    deg = d0_ref[...] + d1_ref[...] + 1.0
    dis = lax.rsqrt(deg)
    dis_ref[...] = dis
    row = (pl.program_id(0) * _BR
           + lax.broadcasted_iota(jnp.int32, (_BR, 1), 0))
    xd_ref[...] = jnp.where(row < N, x_ref[...] * dis, 0.0)


_s1_call = pl.pallas_call(
    _s1_body,
    grid=(_GRID,),
    in_specs=[
        pl.BlockSpec((_BR, D_IN), lambda i: (i, 0)),
        pl.BlockSpec((_BR, 1), lambda i: (i, 0)),
        pl.BlockSpec((_BR, 1), lambda i: (i, 0)),
    ],
    out_specs=[
        pl.BlockSpec((_BR, D_IN), lambda i: (i, 0)),
        pl.BlockSpec((_BR, 1), lambda i: (i, 0)),
    ],
    out_shape=[
        jax.ShapeDtypeStruct((R, D_IN), F32),
        jax.ShapeDtypeStruct((R, 1), F32),
    ],
)


def _t2_body(p0_ref, p1_ref, xd_ref, dis_ref, b1_ref, w1_ref, w2_ref,
             hrelu_ref, ht2_ref):
    dis = dis_ref[...]
    u = p0_ref[...] + p1_ref[...] + xd_ref[...]
    h1 = jnp.dot(u, w1_ref[...], preferred_element_type=F32)
    out1 = h1 * dis + b1_ref[...]
    hr = jnp.maximum(out1, 0.0)
    hrelu_ref[...] = hr
    ht2_ref[...] = jnp.dot(hr, w2_ref[...],
                           preferred_element_type=F32) * dis


_t2_call = pl.pallas_call(
    _t2_body,
    grid=(_GRID,),
    in_specs=[
        pl.BlockSpec((_BR, D_IN), lambda i: (i, 0)),
        pl.BlockSpec((_BR, D_IN), lambda i: (i, 0)),
        pl.BlockSpec((_BR, D_IN), lambda i: (i, 0)),
        pl.BlockSpec((_BR, 1), lambda i: (i, 0)),
        pl.BlockSpec((1, D_HID), lambda i: (0, 0)),
        pl.BlockSpec((D_IN, D_HID), lambda i: (0, 0)),
        pl.BlockSpec((D_HID, D_OUT), lambda i: (0, 0)),
    ],
    out_specs=[
        pl.BlockSpec((_BR, D_HID), lambda i: (i, 0)),
        pl.BlockSpec((_BR, D_OUT), lambda i: (i, 0)),
    ],
    out_shape=[
        jax.ShapeDtypeStruct((N, D_HID), F32),
        jax.ShapeDtypeStruct((R, D_OUT), F32),
    ],
)


def _k3_body(a0_ref, a1_ref, ht2_ref, dis_ref, b2_ref, out_ref):
    s = a0_ref[...] + a1_ref[...] + ht2_ref[...]
    out_ref[...] = s * dis_ref[...] + b2_ref[...]


_k3_call = pl.pallas_call(
    _k3_body,
    grid=(_GRID,),
    in_specs=[
        pl.BlockSpec((_BR, D_OUT), lambda i: (i, 0)),
        pl.BlockSpec((_BR, D_OUT), lambda i: (i, 0)),
        pl.BlockSpec((_BR, D_OUT), lambda i: (i, 0)),
        pl.BlockSpec((_BR, 1), lambda i: (i, 0)),
        pl.BlockSpec((1, D_OUT), lambda i: (0, 0)),
    ],
    out_specs=pl.BlockSpec((_BR, D_OUT), lambda i: (i, 0)),
    out_shape=jax.ShapeDtypeStruct((N, D_OUT), F32),
)


def kernel(x, edge_index, W1, b1, W2, b2):
    pad_e = EPAD - E
    spread = (N + jnp.arange(pad_e, dtype=edge_index.dtype) % (R - N))
    src = jnp.concatenate([edge_index[0], spread])
    dst = jnp.concatenate([edge_index[1], spread])
    src2 = src.reshape(EROWS, CHUNK)
    dst2 = dst.reshape(EROWS, CHUNK)
    deg0, deg1 = _deg_call(dst2)
    xd, dis = _s1_call(x, deg0.reshape(R, 1), deg1.reshape(R, 1))
    p0, p1 = _agg2_call(xd, src, dst)
    hrelu, ht2 = _t2_call(p0, p1, xd, dis, b1.reshape(1, -1), W1, W2)
    q0, q1 = _agg2_call(ht2, src, dst)
    out2 = _k3_call(q0, q1, ht2, dis, b2.reshape(1, -1))
    return out2, hrelu


# speedup vs baseline: 35.2522x; 1.0380x over previous
import jax
import jax.numpy as jnp
import numpy as np
from jax import lax
from jax.experimental import pallas as pl
from jax.experimental.pallas import tpu as pltpu
from jax.experimental.pallas import tpu_sc as plsc

N = 10000
D_IN = 128
D_HID = 256
D_OUT = 128
E = 320000
R = 10240
CHUNK = 128
NC, NS = 2, 16
EROWS = 2560
ERR = E // CHUNK
EPAD = EROWS * CHUNK
NBUF = 4
HALF = R // 2
F32 = jnp.float32

_MESH = plsc.VectorSubcoreMesh(core_axis_name="c", subcore_axis_name="s")


def _deg_body(ej_hbm, deg0_hbm, deg1_hbm, idx_v, acc_v, red_v, tmp_v,
              stage_sh):
    c = lax.axis_index("c")
    s = lax.axis_index("s")
    wid = c * NS + s
    base_rows = ERR // (NC * NS)
    extra = ERR % (NC * NS)
    nrows = base_rows + jnp.where(wid < extra, 1, 0)
    lanes = lax.iota(jnp.int32, 16)
    lane_off = lanes * HALF
    ones = jnp.ones((16,), F32)
    pltpu.sync_copy(ej_hbm.at[1, pl.ds(wid * base_rows * CHUNK,
                                       base_rows * CHUNK)],
                    idx_v.at[pl.ds(0, base_rows * CHUNK)])

    @pl.when(wid < extra)
    def _():
        pltpu.sync_copy(
            ej_hbm.at[1, pl.ds((base_rows * NC * NS + wid) * CHUNK, CHUNK)],
            idx_v.at[pl.ds(base_rows * CHUNK, CHUNK)])

    for p in range(2):
        lo = p * HALF

        def zero_acc(i, _):
            for u in range(8):
                acc_v[pl.ds((i * 8 + u) * 16, 16)] = jnp.zeros((16,), F32)
            return 0
        lax.fori_loop(0, (16 * HALF) // 128, zero_acc, 0)

        def row_body(i, _):
            for k in range(CHUNK // 16):
                v = idx_v[pl.ds(i * CHUNK + k * 16, 16)]
                rel = v - lo
                m = (rel >= 0) & (rel < HALF)
                rel_c = jnp.clip(rel, 0, HALF - 1)
                plsc.addupdate_scatter(acc_v, [rel_c + lane_off], ones,
                                       mask=m)
            return 0
        lax.fori_loop(0, nrows, row_body, 0)

        def red_body(j, _):
            t = acc_v[pl.ds(j * 16, 16)]
            for l in range(1, 16):
                t = t + acc_v[pl.ds(l * HALF + j * 16, 16)]
            red_v[pl.ds(lo + j * 16, 16)] = t
            return 0
        lax.fori_loop(0, HALF // 16, red_body, 0)

    pltpu.sync_copy(red_v, stage_sh.at[s])
    plsc.subcore_barrier()
    rows = R // NS
    pltpu.sync_copy(stage_sh.at[:, pl.ds(s * rows, rows)], tmp_v)

    def add_body(j, _):
        acc = tmp_v[0, pl.ds(j * 16, 16)]
        for l in range(1, NS):
            acc = acc + tmp_v[l, pl.ds(j * 16, 16)]
        red_v[pl.ds(j * 16, 16)] = acc
        return 0
    lax.fori_loop(0, rows // 16, add_body, 0)

    @pl.when(c == 0)
    def _():
        pltpu.sync_copy(red_v.at[pl.ds(0, rows)],
                        deg0_hbm.at[pl.ds(s * rows, rows)])

    @pl.when(c == 1)
    def _():
        pltpu.sync_copy(red_v.at[pl.ds(0, rows)],
                        deg1_hbm.at[pl.ds(s * rows, rows)])


_deg_call = pl.kernel(
    _deg_body,
    name='degk',
    out_type=[jax.ShapeDtypeStruct((R,), F32),
              jax.ShapeDtypeStruct((R,), F32)],
    mesh=_MESH,
    compiler_params=pltpu.CompilerParams(needs_layout_passes=False),
    scratch_types=[
        pltpu.VMEM(((ERR // (NC * NS) + 1) * CHUNK,), jnp.int32),
        pltpu.VMEM((16 * HALF,), F32),
        pltpu.VMEM((R,), F32),
        pltpu.VMEM((NS, R // NS), F32),
        pltpu.VMEM_SHARED((NS, R), F32),
    ],
)


def _fill_zeros2d(ref, rows, cols):
    def body(i, _):
        for j in range(cols // 16):
            ref[i, pl.ds(j * 16, 16)] = jnp.zeros((16,), F32)
        return 0
    lax.fori_loop(0, rows, body, 0)


def _idx_wait(src_hbm, sb, db, semi):
    pltpu.make_async_copy(src_hbm.at[pl.ds(0, CHUNK)], sb, semi).wait()
    pltpu.make_async_copy(src_hbm.at[pl.ds(0, CHUNK)], db, semi).wait()


def _agg_body_common(ht_hbm, src_hbm, dst_hbm, acc_sh, gbufs, sbufs, dbufs,
                     semg, semi, s, row0, nrows):
    _fill_zeros2d(gbufs[0], CHUNK, gbufs[0].shape[1])
    rows = R // NS
    zdescs = [pltpu.make_async_copy(
        gbufs[0], acc_sh.at[pl.ds(s * rows + k * CHUNK, CHUNK)], semg[0])
        for k in range(rows // CHUNK)]
    for d in zdescs:
        d.start()
    for d in zdescs:
        d.wait()
    plsc.subcore_barrier()

    for tslot in range(4):
        pltpu.async_copy(src_hbm.at[pl.ds((row0 + tslot) * CHUNK, CHUNK)],
                         sbufs[tslot], semi[tslot])
        pltpu.async_copy(dst_hbm.at[pl.ds((row0 + tslot) * CHUNK, CHUNK)],
                         dbufs[tslot], semi[tslot])
    for bg in range(2):
        _idx_wait(src_hbm, sbufs[bg], dbufs[bg], semi[bg])
        pltpu.async_copy(ht_hbm.at[sbufs[bg]], gbufs[bg], semg[bg])

    nsteps = nrows // 4

    def step(g, _):
        for b4 in range(4):
            i = g * 4 + b4
            gi = b4 % 2
            s2 = (b4 + 2) % 4
            pltpu.make_async_copy(ht_hbm.at[sbufs[b4]], gbufs[gi],
                                  semg[gi]).wait()
            pltpu.sync_copy(gbufs[gi], acc_sh.at[dbufs[b4]], add=True)
            @pl.when(g < nsteps - 1)
            def _():
                pltpu.async_copy(
                    src_hbm.at[pl.ds((row0 + i + 4) * CHUNK, CHUNK)],
                    sbufs[b4], semi[b4])
                pltpu.async_copy(
                    dst_hbm.at[pl.ds((row0 + i + 4) * CHUNK, CHUNK)],
                    dbufs[b4], semi[b4])
            if b4 < 2:
                _idx_wait(src_hbm, sbufs[s2], dbufs[s2], semi[s2])
                pltpu.async_copy(ht_hbm.at[sbufs[s2]], gbufs[gi],
                                 semg[gi])
            else:
                @pl.when(g < nsteps - 1)
                def _():
                    _idx_wait(src_hbm, sbufs[s2], dbufs[s2], semi[s2])
                    pltpu.async_copy(ht_hbm.at[sbufs[s2]], gbufs[gi],
                                     semg[gi])
        return 0
    lax.fori_loop(0, nsteps, step, 0)


def _agg_epilogue(acc_sh, out_hbm, s):
    rows = R // NS
    pltpu.sync_copy(acc_sh.at[pl.ds(s * rows, rows)],
                    out_hbm.at[pl.ds(s * rows, rows)])


def _agg_scratch(dsc):
    return [
        pltpu.VMEM((CHUNK, dsc), F32),
        pltpu.VMEM((CHUNK, dsc), F32),
        pltpu.VMEM((CHUNK,), jnp.int32),
        pltpu.VMEM((CHUNK,), jnp.int32),
        pltpu.VMEM((CHUNK,), jnp.int32),
        pltpu.VMEM((CHUNK,), jnp.int32),
        pltpu.VMEM((CHUNK,), jnp.int32),
        pltpu.VMEM((CHUNK,), jnp.int32),
        pltpu.VMEM((CHUNK,), jnp.int32),
        pltpu.VMEM((CHUNK,), jnp.int32),
        pltpu.VMEM_SHARED((R, dsc), F32),
        pltpu.SemaphoreType.DMA,
        pltpu.SemaphoreType.DMA,
        pltpu.SemaphoreType.DMA,
        pltpu.SemaphoreType.DMA,
        pltpu.SemaphoreType.DMA,
        pltpu.SemaphoreType.DMA,
    ]


def _agg2_body(ht_hbm, src_hbm, dst_hbm, agg0_hbm, agg1_hbm,
               gb0, gb1, sb0, sb1, sb2, sb3, db0, db1, db2, db3, acc_sh,
               smg0, smg1, smi0, smi1, smi2, smi3):
    c = lax.axis_index("c")
    s = lax.axis_index("s")
    nrows = EROWS // (NC * NS)
    _agg_body_common(ht_hbm, src_hbm, dst_hbm, acc_sh,
                     (gb0, gb1), (sb0, sb1, sb2, sb3),
                     (db0, db1, db2, db3), (smg0, smg1),
                     (smi0, smi1, smi2, smi3), s,
                     (c * NS + s) * nrows, nrows)
    plsc.subcore_barrier()

    @pl.when(c == 0)
    def _():
        _agg_epilogue(acc_sh, agg0_hbm, s)

    @pl.when(c == 1)
    def _():
        _agg_epilogue(acc_sh, agg1_hbm, s)


_agg2_call = pl.kernel(
    _agg2_body,
    name='agg2k',
    out_type=[jax.ShapeDtypeStruct((R, D_OUT), F32),
              jax.ShapeDtypeStruct((R, D_OUT), F32)],
    mesh=_MESH,
    scratch_types=_agg_scratch(D_OUT),
)


_BR = 1024
_GRID = R // _BR


def _s1_body(x_ref, d0_ref, d1_ref, xd_ref, dis_ref):
    deg = d0_ref[...] + d1_ref[...] + 1.0
    dis = lax.rsqrt(deg)
    dis_ref[...] = dis
    row = (pl.program_id(0) * _BR
           + lax.broadcasted_iota(jnp.int32, (_BR, 1), 0))
    xd_ref[...] = jnp.where(row < N, x_ref[...] * dis, 0.0)


_s1_call = pl.pallas_call(
    _s1_body,
    grid=(_GRID,),
    in_specs=[
        pl.BlockSpec((_BR, D_IN), lambda i: (i, 0)),
        pl.BlockSpec((_BR, 1), lambda i: (i, 0)),
        pl.BlockSpec((_BR, 1), lambda i: (i, 0)),
    ],
    out_specs=[
        pl.BlockSpec((_BR, D_IN), lambda i: (i, 0)),
        pl.BlockSpec((_BR, 1), lambda i: (i, 0)),
    ],
    out_shape=[
        jax.ShapeDtypeStruct((R, D_IN), F32),
        jax.ShapeDtypeStruct((R, 1), F32),
    ],
)


def _t2_body(p0_ref, p1_ref, xd_ref, dis_ref, b1_ref, w1_ref, w2_ref,
             hrelu_ref, ht2_ref):
    dis = dis_ref[...]
    u = p0_ref[...] + p1_ref[...] + xd_ref[...]
    h1 = jnp.dot(u, w1_ref[...], preferred_element_type=F32)
    out1 = h1 * dis + b1_ref[...]
    hr = jnp.maximum(out1, 0.0)
    hrelu_ref[...] = hr
    ht2_ref[...] = jnp.dot(hr, w2_ref[...],
                           preferred_element_type=F32) * dis


_t2_call = pl.pallas_call(
    _t2_body,
    grid=(_GRID,),
    in_specs=[
        pl.BlockSpec((_BR, D_IN), lambda i: (i, 0)),
        pl.BlockSpec((_BR, D_IN), lambda i: (i, 0)),
        pl.BlockSpec((_BR, D_IN), lambda i: (i, 0)),
        pl.BlockSpec((_BR, 1), lambda i: (i, 0)),
        pl.BlockSpec((1, D_HID), lambda i: (0, 0)),
        pl.BlockSpec((D_IN, D_HID), lambda i: (0, 0)),
        pl.BlockSpec((D_HID, D_OUT), lambda i: (0, 0)),
    ],
    out_specs=[
        pl.BlockSpec((_BR, D_HID), lambda i: (i, 0)),
        pl.BlockSpec((_BR, D_OUT), lambda i: (i, 0)),
    ],
    out_shape=[
        jax.ShapeDtypeStruct((N, D_HID), F32),
        jax.ShapeDtypeStruct((R, D_OUT), F32),
    ],
)


def _k3_body(a0_ref, a1_ref, ht2_ref, dis_ref, b2_ref, out_ref):
    s = a0_ref[...] + a1_ref[...] + ht2_ref[...]
    out_ref[...] = s * dis_ref[...] + b2_ref[...]


_k3_call = pl.pallas_call(
    _k3_body,
    grid=(_GRID,),
    in_specs=[
        pl.BlockSpec((_BR, D_OUT), lambda i: (i, 0)),
        pl.BlockSpec((_BR, D_OUT), lambda i: (i, 0)),
        pl.BlockSpec((_BR, D_OUT), lambda i: (i, 0)),
        pl.BlockSpec((_BR, 1), lambda i: (i, 0)),
        pl.BlockSpec((1, D_OUT), lambda i: (0, 0)),
    ],
    out_specs=pl.BlockSpec((_BR, D_OUT), lambda i: (i, 0)),
    out_shape=jax.ShapeDtypeStruct((N, D_OUT), F32),
)


def kernel(x, edge_index, W1, b1, W2, b2):
    pad_e = EPAD - E
    spread = jnp.asarray(N + np.arange(pad_e) % (R - N), dtype=jnp.int32)
    src = jnp.concatenate([edge_index[0], spread])
    dst = jnp.concatenate([edge_index[1], spread])
    deg0, deg1 = _deg_call(edge_index)
    xd, dis = _s1_call(x, deg0.reshape(R, 1), deg1.reshape(R, 1))
    p0, p1 = _agg2_call(xd, src, dst)
    hrelu, ht2 = _t2_call(p0, p1, xd, dis, b1.reshape(1, -1), W1, W2)
    q0, q1 = _agg2_call(ht2, src, dst)
    out2 = _k3_call(q0, q1, ht2, dis, b2.reshape(1, -1))
    return out2, hrelu


# speedup vs baseline: 36.7918x; 1.0437x over previous
import jax
import jax.numpy as jnp
import numpy as np
from jax import lax
from jax.experimental import pallas as pl
from jax.experimental.pallas import tpu as pltpu
from jax.experimental.pallas import tpu_sc as plsc

N = 10000
D_IN = 128
D_HID = 256
D_OUT = 128
E = 320000
R = 10240
CHUNK = 128
NC, NS = 2, 16
EROWS = 2560
ERR = E // CHUNK
EPAD = EROWS * CHUNK
NBUF = 4
HALF = R // 2
F32 = jnp.float32

_MESH = plsc.VectorSubcoreMesh(core_axis_name="c", subcore_axis_name="s")


def _deg_body(ej_hbm, deg0_hbm, deg1_hbm, idx_v, acc_v, red_v, tmp_v,
              stage_sh):
    c = lax.axis_index("c")
    s = lax.axis_index("s")
    base_rows = ERR // NS
    extra = ERR % NS
    nrows = base_rows + jnp.where(s < extra, 1, 0)
    lanes = lax.iota(jnp.int32, 16)
    lane_off = lanes * HALF
    ones = jnp.ones((16,), F32)
    lo = c * HALF
    pltpu.sync_copy(ej_hbm.at[1, pl.ds(s * base_rows * CHUNK,
                                       base_rows * CHUNK)],
                    idx_v.at[pl.ds(0, base_rows * CHUNK)])

    @pl.when(s < extra)
    def _():
        pltpu.sync_copy(
            ej_hbm.at[1, pl.ds((base_rows * NS + s) * CHUNK, CHUNK)],
            idx_v.at[pl.ds(base_rows * CHUNK, CHUNK)])

    def zero_acc(i, _):
        for u in range(8):
            acc_v[pl.ds((i * 8 + u) * 16, 16)] = jnp.zeros((16,), F32)
        return 0
    lax.fori_loop(0, (16 * HALF) // 128, zero_acc, 0)

    def row_body(i, _):
        for k in range(CHUNK // 16):
            v = idx_v[pl.ds(i * CHUNK + k * 16, 16)]
            rel = v - lo
            m = (rel >= 0) & (rel < HALF)
            rel_c = jnp.clip(rel, 0, HALF - 1)
            plsc.addupdate_scatter(acc_v, [rel_c + lane_off], ones,
                                   mask=m)
        return 0
    lax.fori_loop(0, nrows, row_body, 0)

    def red_body(j, _):
        t = acc_v[pl.ds(j * 16, 16)]
        for l in range(1, 16):
            t = t + acc_v[pl.ds(l * HALF + j * 16, 16)]
        red_v[pl.ds(j * 16, 16)] = t
        return 0
    lax.fori_loop(0, HALF // 16, red_body, 0)

    pltpu.sync_copy(red_v, stage_sh.at[s])
    plsc.subcore_barrier()
    rows = HALF // 8

    @pl.when(s < 8)
    def _():
        pltpu.sync_copy(stage_sh.at[:, pl.ds(s * rows, rows)], tmp_v)

        def add_body(j, _):
            acc = tmp_v[0, pl.ds(j * 16, 16)]
            for l in range(1, NS):
                acc = acc + tmp_v[l, pl.ds(j * 16, 16)]
            red_v[pl.ds(j * 16, 16)] = acc
            return 0
        lax.fori_loop(0, rows // 16, add_body, 0)

        @pl.when(c == 0)
        def _():
            pltpu.sync_copy(red_v.at[pl.ds(0, rows)],
                            deg0_hbm.at[pl.ds(s * rows, rows)])

        @pl.when(c == 1)
        def _():
            pltpu.sync_copy(red_v.at[pl.ds(0, rows)],
                            deg1_hbm.at[pl.ds(s * rows, rows)])


_deg_call = pl.kernel(
    _deg_body,
    name='degk',
    out_type=[jax.ShapeDtypeStruct((HALF,), F32),
              jax.ShapeDtypeStruct((HALF,), F32)],
    mesh=_MESH,
    compiler_params=pltpu.CompilerParams(needs_layout_passes=False),
    scratch_types=[
        pltpu.VMEM(((ERR // NS + 1) * CHUNK,), jnp.int32),
        pltpu.VMEM((16 * HALF,), F32),
        pltpu.VMEM((HALF,), F32),
        pltpu.VMEM((NS, HALF // 8), F32),
        pltpu.VMEM_SHARED((NS, HALF), F32),
    ],
)


def _fill_zeros2d(ref, rows, cols):
    def body(i, _):
        for j in range(cols // 16):
            ref[i, pl.ds(j * 16, 16)] = jnp.zeros((16,), F32)
        return 0
    lax.fori_loop(0, rows, body, 0)


def _idx_wait(src_hbm, sb, db, semi):
    pltpu.make_async_copy(src_hbm.at[pl.ds(0, CHUNK)], sb, semi).wait()
    pltpu.make_async_copy(src_hbm.at[pl.ds(0, CHUNK)], db, semi).wait()


def _agg_body_common(ht_hbm, src_hbm, dst_hbm, acc_sh, gbufs, sbufs, dbufs,
                     semg, semi, s, row0, nrows):
    _fill_zeros2d(gbufs[0], CHUNK, gbufs[0].shape[1])
    rows = R // NS
    zdescs = [pltpu.make_async_copy(
        gbufs[0], acc_sh.at[pl.ds(s * rows + k * CHUNK, CHUNK)], semg[0])
        for k in range(rows // CHUNK)]
    for d in zdescs:
        d.start()
    for d in zdescs:
        d.wait()
    plsc.subcore_barrier()

    for tslot in range(4):
        pltpu.async_copy(src_hbm.at[pl.ds((row0 + tslot) * CHUNK, CHUNK)],
                         sbufs[tslot], semi[tslot])
        pltpu.async_copy(dst_hbm.at[pl.ds((row0 + tslot) * CHUNK, CHUNK)],
                         dbufs[tslot], semi[tslot])
    for bg in range(2):
        _idx_wait(src_hbm, sbufs[bg], dbufs[bg], semi[bg])
        pltpu.async_copy(ht_hbm.at[sbufs[bg]], gbufs[bg], semg[bg])

    nsteps = nrows // 4

    def step(g, _):
        for b4 in range(4):
            i = g * 4 + b4
            gi = b4 % 2
            s2 = (b4 + 2) % 4
            pltpu.make_async_copy(ht_hbm.at[sbufs[b4]], gbufs[gi],
                                  semg[gi]).wait()
            pltpu.sync_copy(gbufs[gi], acc_sh.at[dbufs[b4]], add=True)
            @pl.when(g < nsteps - 1)
            def _():
                pltpu.async_copy(
                    src_hbm.at[pl.ds((row0 + i + 4) * CHUNK, CHUNK)],
                    sbufs[b4], semi[b4])
                pltpu.async_copy(
                    dst_hbm.at[pl.ds((row0 + i + 4) * CHUNK, CHUNK)],
                    dbufs[b4], semi[b4])
            if b4 < 2:
                _idx_wait(src_hbm, sbufs[s2], dbufs[s2], semi[s2])
                pltpu.async_copy(ht_hbm.at[sbufs[s2]], gbufs[gi],
                                 semg[gi])
            else:
                @pl.when(g < nsteps - 1)
                def _():
                    _idx_wait(src_hbm, sbufs[s2], dbufs[s2], semi[s2])
                    pltpu.async_copy(ht_hbm.at[sbufs[s2]], gbufs[gi],
                                     semg[gi])
        return 0
    lax.fori_loop(0, nsteps, step, 0)


def _agg_epilogue(acc_sh, out_hbm, s):
    rows = R // NS
    pltpu.sync_copy(acc_sh.at[pl.ds(s * rows, rows)],
                    out_hbm.at[pl.ds(s * rows, rows)])


def _agg_scratch(dsc):
    return [
        pltpu.VMEM((CHUNK, dsc), F32),
        pltpu.VMEM((CHUNK, dsc), F32),
        pltpu.VMEM((CHUNK,), jnp.int32),
        pltpu.VMEM((CHUNK,), jnp.int32),
        pltpu.VMEM((CHUNK,), jnp.int32),
        pltpu.VMEM((CHUNK,), jnp.int32),
        pltpu.VMEM((CHUNK,), jnp.int32),
        pltpu.VMEM((CHUNK,), jnp.int32),
        pltpu.VMEM((CHUNK,), jnp.int32),
        pltpu.VMEM((CHUNK,), jnp.int32),
        pltpu.VMEM_SHARED((R, dsc), F32),
        pltpu.SemaphoreType.DMA,
        pltpu.SemaphoreType.DMA,
        pltpu.SemaphoreType.DMA,
        pltpu.SemaphoreType.DMA,
        pltpu.SemaphoreType.DMA,
        pltpu.SemaphoreType.DMA,
    ]


def _agg2_body(ht_hbm, src_hbm, dst_hbm, agg0_hbm, agg1_hbm,
               gb0, gb1, sb0, sb1, sb2, sb3, db0, db1, db2, db3, acc_sh,
               smg0, smg1, smi0, smi1, smi2, smi3):
    c = lax.axis_index("c")
    s = lax.axis_index("s")
    nrows = EROWS // (NC * NS)
    _agg_body_common(ht_hbm, src_hbm, dst_hbm, acc_sh,
                     (gb0, gb1), (sb0, sb1, sb2, sb3),
                     (db0, db1, db2, db3), (smg0, smg1),
                     (smi0, smi1, smi2, smi3), s,
                     (c * NS + s) * nrows, nrows)
    plsc.subcore_barrier()

    @pl.when(c == 0)
    def _():
        _agg_epilogue(acc_sh, agg0_hbm, s)

    @pl.when(c == 1)
    def _():
        _agg_epilogue(acc_sh, agg1_hbm, s)


_agg2_call = pl.kernel(
    _agg2_body,
    name='agg2k',
    out_type=[jax.ShapeDtypeStruct((R, D_OUT), F32),
              jax.ShapeDtypeStruct((R, D_OUT), F32)],
    mesh=_MESH,
    scratch_types=_agg_scratch(D_OUT),
)


_BR = 1024
_GRID = R // _BR


def _s1_body(x_ref, dg_ref, xd_ref, dis_ref):
    deg = dg_ref[...] + 1.0
    dis = lax.rsqrt(deg)
    dis_ref[...] = dis
    row = (pl.program_id(0) * _BR
           + lax.broadcasted_iota(jnp.int32, (_BR, 1), 0))
    xd_ref[...] = jnp.where(row < N, x_ref[...] * dis, 0.0)


_s1_call = pl.pallas_call(
    _s1_body,
    grid=(_GRID,),
    in_specs=[
        pl.BlockSpec((_BR, D_IN), lambda i: (i, 0)),
        pl.BlockSpec((_BR, 1), lambda i: (i, 0)),
    ],
    out_specs=[
        pl.BlockSpec((_BR, D_IN), lambda i: (i, 0)),
        pl.BlockSpec((_BR, 1), lambda i: (i, 0)),
    ],
    out_shape=[
        jax.ShapeDtypeStruct((R, D_IN), F32),
        jax.ShapeDtypeStruct((R, 1), F32),
    ],
)


def _t2_body(p0_ref, p1_ref, xd_ref, dis_ref, b1_ref, w1_ref, w2_ref,
             hrelu_ref, ht2_ref):
    dis = dis_ref[...]
    u = p0_ref[...] + p1_ref[...] + xd_ref[...]
    h1 = jnp.dot(u, w1_ref[...], preferred_element_type=F32)
    out1 = h1 * dis + b1_ref[...]
    hr = jnp.maximum(out1, 0.0)
    hrelu_ref[...] = hr
    ht2_ref[...] = jnp.dot(hr, w2_ref[...],
                           preferred_element_type=F32) * dis


_t2_call = pl.pallas_call(
    _t2_body,
    grid=(_GRID,),
    in_specs=[
        pl.BlockSpec((_BR, D_IN), lambda i: (i, 0)),
        pl.BlockSpec((_BR, D_IN), lambda i: (i, 0)),
        pl.BlockSpec((_BR, D_IN), lambda i: (i, 0)),
        pl.BlockSpec((_BR, 1), lambda i: (i, 0)),
        pl.BlockSpec((1, D_HID), lambda i: (0, 0)),
        pl.BlockSpec((D_IN, D_HID), lambda i: (0, 0)),
        pl.BlockSpec((D_HID, D_OUT), lambda i: (0, 0)),
    ],
    out_specs=[
        pl.BlockSpec((_BR, D_HID), lambda i: (i, 0)),
        pl.BlockSpec((_BR, D_OUT), lambda i: (i, 0)),
    ],
    out_shape=[
        jax.ShapeDtypeStruct((N, D_HID), F32),
        jax.ShapeDtypeStruct((R, D_OUT), F32),
    ],
)


def _k3_body(a0_ref, a1_ref, ht2_ref, dis_ref, b2_ref, out_ref):
    s = a0_ref[...] + a1_ref[...] + ht2_ref[...]
    out_ref[...] = s * dis_ref[...] + b2_ref[...]


_k3_call = pl.pallas_call(
    _k3_body,
    grid=(_GRID,),
    in_specs=[
        pl.BlockSpec((_BR, D_OUT), lambda i: (i, 0)),
        pl.BlockSpec((_BR, D_OUT), lambda i: (i, 0)),
        pl.BlockSpec((_BR, D_OUT), lambda i: (i, 0)),
        pl.BlockSpec((_BR, 1), lambda i: (i, 0)),
        pl.BlockSpec((1, D_OUT), lambda i: (0, 0)),
    ],
    out_specs=pl.BlockSpec((_BR, D_OUT), lambda i: (i, 0)),
    out_shape=jax.ShapeDtypeStruct((N, D_OUT), F32),
)


def kernel(x, edge_index, W1, b1, W2, b2):
    pad_e = EPAD - E
    spread = jnp.asarray(N + np.arange(pad_e) % (R - N), dtype=jnp.int32)
    src = jnp.concatenate([edge_index[0], spread])
    dst = jnp.concatenate([edge_index[1], spread])
    deg0, deg1 = _deg_call(edge_index)
    deg = jnp.concatenate([deg0, deg1]).reshape(R, 1)
    xd, dis = _s1_call(x, deg)
    p0, p1 = _agg2_call(xd, src, dst)
    hrelu, ht2 = _t2_call(p0, p1, xd, dis, b1.reshape(1, -1), W1, W2)
    q0, q1 = _agg2_call(ht2, src, dst)
    out2 = _k3_call(q0, q1, ht2, dis, b2.reshape(1, -1))
    return out2, hrelu


# speedup vs baseline: 37.5878x; 1.0216x over previous
import jax
import jax.numpy as jnp
import numpy as np
from jax import lax
from jax.experimental import pallas as pl
from jax.experimental.pallas import tpu as pltpu
from jax.experimental.pallas import tpu_sc as plsc

N = 10000
D_IN = 128
D_HID = 256
D_OUT = 128
E = 320000
R = 10240
CHUNK = 128
NC, NS = 2, 16
EROWS = 2560
ERR = E // CHUNK
EPAD = EROWS * CHUNK
NBUF = 4
HALF = R // 2
F32 = jnp.float32

_MESH = plsc.VectorSubcoreMesh(core_axis_name="c", subcore_axis_name="s")


def _deg_body(ej_hbm, deg0_hbm, deg1_hbm, idx_v, acc_v, red_v, tmp_v,
              stage_sh):
    c = lax.axis_index("c")
    s = lax.axis_index("s")
    base_rows = ERR // NS
    extra = ERR % NS
    nrows = base_rows + jnp.where(s < extra, 1, 0)
    lanes = lax.iota(jnp.int32, 16)
    lane_off = lanes * HALF
    ones = jnp.ones((16,), F32)
    lo = c * HALF
    pltpu.sync_copy(ej_hbm.at[1, pl.ds(s * base_rows * CHUNK,
                                       base_rows * CHUNK)],
                    idx_v.at[pl.ds(0, base_rows * CHUNK)])

    @pl.when(s < extra)
    def _():
        pltpu.sync_copy(
            ej_hbm.at[1, pl.ds((base_rows * NS + s) * CHUNK, CHUNK)],
            idx_v.at[pl.ds(base_rows * CHUNK, CHUNK)])

    def zero_acc(i, _):
        for u in range(8):
            acc_v[pl.ds((i * 8 + u) * 16, 16)] = jnp.zeros((16,), F32)
        return 0
    lax.fori_loop(0, (16 * HALF) // 128, zero_acc, 0)

    def row_body(i, _):
        for k in range(CHUNK // 16):
            v = idx_v[pl.ds(i * CHUNK + k * 16, 16)]
            rel = v - lo
            m = (rel >= 0) & (rel < HALF)
            rel_c = jnp.clip(rel, 0, HALF - 1)
            plsc.addupdate_scatter(acc_v, [rel_c + lane_off], ones,
                                   mask=m)
        return 0
    lax.fori_loop(0, nrows, row_body, 0)

    def red_body(j, _):
        t = acc_v[pl.ds(j * 16, 16)]
        for l in range(1, 16):
            t = t + acc_v[pl.ds(l * HALF + j * 16, 16)]
        red_v[pl.ds(j * 16, 16)] = t
        return 0
    lax.fori_loop(0, HALF // 16, red_body, 0)

    pltpu.sync_copy(red_v, stage_sh.at[s])
    plsc.subcore_barrier()
    rows = HALF // 8

    @pl.when(s < 8)
    def _():
        pltpu.sync_copy(stage_sh.at[:, pl.ds(s * rows, rows)], tmp_v)

        def add_body(j, _):
            acc = tmp_v[0, pl.ds(j * 16, 16)]
            for l in range(1, NS):
                acc = acc + tmp_v[l, pl.ds(j * 16, 16)]
            red_v[pl.ds(j * 16, 16)] = acc
            return 0
        lax.fori_loop(0, rows // 16, add_body, 0)

        @pl.when(c == 0)
        def _():
            pltpu.sync_copy(red_v.at[pl.ds(0, rows)],
                            deg0_hbm.at[pl.ds(s * rows, rows)])

        @pl.when(c == 1)
        def _():
            pltpu.sync_copy(red_v.at[pl.ds(0, rows)],
                            deg1_hbm.at[pl.ds(s * rows, rows)])


_deg_call = pl.kernel(
    _deg_body,
    name='degk',
    out_type=[jax.ShapeDtypeStruct((HALF,), F32),
              jax.ShapeDtypeStruct((HALF,), F32)],
    mesh=_MESH,
    compiler_params=pltpu.CompilerParams(needs_layout_passes=False),
    scratch_types=[
        pltpu.VMEM(((ERR // NS + 1) * CHUNK,), jnp.int32),
        pltpu.VMEM((16 * HALF,), F32),
        pltpu.VMEM((HALF,), F32),
        pltpu.VMEM((NS, HALF // 8), F32),
        pltpu.VMEM_SHARED((NS, HALF), F32),
    ],
)


def _fill_zeros2d(ref, rows, cols):
    def body(i, _):
        for j in range(cols // 16):
            ref[i, pl.ds(j * 16, 16)] = jnp.zeros((16,), F32)
        return 0
    lax.fori_loop(0, rows, body, 0)


def _idx_wait(src_hbm, sb, db, semi):
    pltpu.make_async_copy(src_hbm.at[pl.ds(0, CHUNK)], sb, semi).wait()
    pltpu.make_async_copy(src_hbm.at[pl.ds(0, CHUNK)], db, semi).wait()


def _agg_body_common(ht_hbm, src_hbm, dst_hbm, acc_sh, gbufs, sbufs, dbufs,
                     semg, semi, s, row0, nrows):
    _fill_zeros2d(gbufs[0], CHUNK, gbufs[0].shape[1])
    rows = R // NS
    zdescs = [pltpu.make_async_copy(
        gbufs[0], acc_sh.at[pl.ds(s * rows + k * CHUNK, CHUNK)], semg[0])
        for k in range(rows // CHUNK)]
    for d in zdescs:
        d.start()
    for d in zdescs:
        d.wait()
    plsc.subcore_barrier()

    for tslot in range(4):
        pltpu.async_copy(src_hbm.at[pl.ds((row0 + tslot) * CHUNK, CHUNK)],
                         sbufs[tslot], semi[tslot])
        pltpu.async_copy(dst_hbm.at[pl.ds((row0 + tslot) * CHUNK, CHUNK)],
                         dbufs[tslot], semi[tslot])
    for bg in range(2):
        _idx_wait(src_hbm, sbufs[bg], dbufs[bg], semi[bg])
        pltpu.async_copy(ht_hbm.at[sbufs[bg]], gbufs[bg], semg[bg])

    nsteps = nrows // 4

    def step(g, _):
        for b4 in range(4):
            i = g * 4 + b4
            gi = b4 % 2
            s2 = (b4 + 2) % 4
            pltpu.make_async_copy(ht_hbm.at[sbufs[b4]], gbufs[gi],
                                  semg[gi]).wait()
            pltpu.sync_copy(gbufs[gi], acc_sh.at[dbufs[b4]], add=True)
            @pl.when(g < nsteps - 1)
            def _():
                pltpu.async_copy(
                    src_hbm.at[pl.ds((row0 + i + 4) * CHUNK, CHUNK)],
                    sbufs[b4], semi[b4])
                pltpu.async_copy(
                    dst_hbm.at[pl.ds((row0 + i + 4) * CHUNK, CHUNK)],
                    dbufs[b4], semi[b4])
            if b4 < 2:
                _idx_wait(src_hbm, sbufs[s2], dbufs[s2], semi[s2])
                pltpu.async_copy(ht_hbm.at[sbufs[s2]], gbufs[gi],
                                 semg[gi])
            else:
                @pl.when(g < nsteps - 1)
                def _():
                    _idx_wait(src_hbm, sbufs[s2], dbufs[s2], semi[s2])
                    pltpu.async_copy(ht_hbm.at[sbufs[s2]], gbufs[gi],
                                     semg[gi])
        return 0
    lax.fori_loop(0, nsteps, step, 0)


def _agg_epilogue(acc_sh, out_hbm, s):
    rows = R // NS
    pltpu.sync_copy(acc_sh.at[pl.ds(s * rows, rows)],
                    out_hbm.at[pl.ds(s * rows, rows)])


def _agg_scratch(dsc):
    return [
        pltpu.VMEM((CHUNK, dsc), F32),
        pltpu.VMEM((CHUNK, dsc), F32),
        pltpu.VMEM((CHUNK,), jnp.int32),
        pltpu.VMEM((CHUNK,), jnp.int32),
        pltpu.VMEM((CHUNK,), jnp.int32),
        pltpu.VMEM((CHUNK,), jnp.int32),
        pltpu.VMEM((CHUNK,), jnp.int32),
        pltpu.VMEM((CHUNK,), jnp.int32),
        pltpu.VMEM((CHUNK,), jnp.int32),
        pltpu.VMEM((CHUNK,), jnp.int32),
        pltpu.VMEM_SHARED((R, dsc), F32),
        pltpu.SemaphoreType.DMA,
        pltpu.SemaphoreType.DMA,
        pltpu.SemaphoreType.DMA,
        pltpu.SemaphoreType.DMA,
        pltpu.SemaphoreType.DMA,
        pltpu.SemaphoreType.DMA,
    ]


def _agg2_body(ht_hbm, src_hbm, dst_hbm, agg0_hbm, agg1_hbm,
               gb0, gb1, sb0, sb1, sb2, sb3, db0, db1, db2, db3, acc_sh,
               smg0, smg1, smi0, smi1, smi2, smi3):
    c = lax.axis_index("c")
    s = lax.axis_index("s")
    nrows = EROWS // (NC * NS)
    _agg_body_common(ht_hbm, src_hbm, dst_hbm, acc_sh,
                     (gb0, gb1), (sb0, sb1, sb2, sb3),
                     (db0, db1, db2, db3), (smg0, smg1),
                     (smi0, smi1, smi2, smi3), s,
                     (c * NS + s) * nrows, nrows)
    plsc.subcore_barrier()

    @pl.when(c == 0)
    def _():
        _agg_epilogue(acc_sh, agg0_hbm, s)

    @pl.when(c == 1)
    def _():
        _agg_epilogue(acc_sh, agg1_hbm, s)


_agg2_call = pl.kernel(
    _agg2_body,
    name='agg2k',
    out_type=[jax.ShapeDtypeStruct((R, D_OUT), F32),
              jax.ShapeDtypeStruct((R, D_OUT), F32)],
    mesh=_MESH,
    scratch_types=_agg_scratch(D_OUT),
)


_BR = 2048
_GRID = R // _BR


def _s1_body(x_ref, dg_ref, xd_ref, dis_ref):
    deg = dg_ref[...] + 1.0
    dis = lax.rsqrt(deg)
    dis_ref[...] = dis
    row = (pl.program_id(0) * _BR
           + lax.broadcasted_iota(jnp.int32, (_BR, 1), 0))
    xd_ref[...] = jnp.where(row < N, x_ref[...] * dis, 0.0)


_s1_call = pl.pallas_call(
    _s1_body,
    grid=(_GRID,),
    in_specs=[
        pl.BlockSpec((_BR, D_IN), lambda i: (i, 0)),
        pl.BlockSpec((_BR, 1), lambda i: (i, 0)),
    ],
    out_specs=[
        pl.BlockSpec((_BR, D_IN), lambda i: (i, 0)),
        pl.BlockSpec((_BR, 1), lambda i: (i, 0)),
    ],
    out_shape=[
        jax.ShapeDtypeStruct((R, D_IN), F32),
        jax.ShapeDtypeStruct((R, 1), F32),
    ],
)


def _t2_body(p0_ref, p1_ref, xd_ref, dis_ref, b1_ref, w1_ref, w2_ref,
             hrelu_ref, ht2_ref):
    dis = dis_ref[...]
    u = p0_ref[...] + p1_ref[...] + xd_ref[...]
    h1 = jnp.dot(u, w1_ref[...], preferred_element_type=F32)
    out1 = h1 * dis + b1_ref[...]
    hr = jnp.maximum(out1, 0.0)
    hrelu_ref[...] = hr
    ht2_ref[...] = jnp.dot(hr, w2_ref[...],
                           preferred_element_type=F32) * dis


_t2_call = pl.pallas_call(
    _t2_body,
    grid=(_GRID,),
    in_specs=[
        pl.BlockSpec((_BR, D_IN), lambda i: (i, 0)),
        pl.BlockSpec((_BR, D_IN), lambda i: (i, 0)),
        pl.BlockSpec((_BR, D_IN), lambda i: (i, 0)),
        pl.BlockSpec((_BR, 1), lambda i: (i, 0)),
        pl.BlockSpec((1, D_HID), lambda i: (0, 0)),
        pl.BlockSpec((D_IN, D_HID), lambda i: (0, 0)),
        pl.BlockSpec((D_HID, D_OUT), lambda i: (0, 0)),
    ],
    out_specs=[
        pl.BlockSpec((_BR, D_HID), lambda i: (i, 0)),
        pl.BlockSpec((_BR, D_OUT), lambda i: (i, 0)),
    ],
    out_shape=[
        jax.ShapeDtypeStruct((N, D_HID), F32),
        jax.ShapeDtypeStruct((R, D_OUT), F32),
    ],
)


def _k3_body(a0_ref, a1_ref, ht2_ref, dis_ref, b2_ref, out_ref):
    s = a0_ref[...] + a1_ref[...] + ht2_ref[...]
    out_ref[...] = s * dis_ref[...] + b2_ref[...]


_k3_call = pl.pallas_call(
    _k3_body,
    grid=(_GRID,),
    in_specs=[
        pl.BlockSpec((_BR, D_OUT), lambda i: (i, 0)),
        pl.BlockSpec((_BR, D_OUT), lambda i: (i, 0)),
        pl.BlockSpec((_BR, D_OUT), lambda i: (i, 0)),
        pl.BlockSpec((_BR, 1), lambda i: (i, 0)),
        pl.BlockSpec((1, D_OUT), lambda i: (0, 0)),
    ],
    out_specs=pl.BlockSpec((_BR, D_OUT), lambda i: (i, 0)),
    out_shape=jax.ShapeDtypeStruct((N, D_OUT), F32),
)


def kernel(x, edge_index, W1, b1, W2, b2):
    pad_e = EPAD - E
    spread = jnp.asarray(N + np.arange(pad_e) % (R - N), dtype=jnp.int32)
    src = jnp.concatenate([edge_index[0], spread])
    dst = jnp.concatenate([edge_index[1], spread])
    deg0, deg1 = _deg_call(edge_index)
    deg = jnp.concatenate([deg0, deg1]).reshape(R, 1)
    xd, dis = _s1_call(x, deg)
    p0, p1 = _agg2_call(xd, src, dst)
    hrelu, ht2 = _t2_call(p0, p1, xd, dis, b1.reshape(1, -1), W1, W2)
    q0, q1 = _agg2_call(ht2, src, dst)
    out2 = _k3_call(q0, q1, ht2, dis, b2.reshape(1, -1))
    return out2, hrelu
